# Initial kernel scaffold; baseline (speedup 1.0000x reference)
import jax
import jax.numpy as jnp
from jax.experimental import pallas as pl


def kernel(x, edge_index, edge_attr, params):
    raise NotImplementedError("write your pallas kernel here")


# speedup vs baseline: 2.9281x; 2.9281x over previous
import functools

import jax
import jax.numpy as jnp
import numpy as np
from jax import lax
from jax.experimental import pallas as pl
from jax.experimental.pallas import tpu as pltpu
from jax.experimental.pallas import tpu_sc as plsc

N = 10000
E = 320000
D = 128
ED = 16
C = 128

NC = 2
NS = 16
NW = NC * NS
EPW = E // NW
GB = 80
SB = 200
INIT_ROWS = 1000

BE = 1600
INV_SQRT_C = 1.0 / float(np.sqrt(C))
BN_EPS = 1e-5


def _sig(z):
    return 1.0 / (1.0 + jnp.exp(-z))


def _node_tables_body(x_ref, wq, bq, wk, bk, wv, bv, wku1, wm1,
                      q_o, ka_o, kb_o, va_o, vb_o):
    xb = x_ref[...]
    q = jnp.dot(xb, wq[...], preferred_element_type=jnp.float32) + bq[...]
    k = jnp.dot(xb, wk[...], preferred_element_type=jnp.float32) + bk[...]
    v = jnp.dot(xb, wv[...], preferred_element_type=jnp.float32) + bv[...]
    q_o[...] = q
    ka_o[...] = jnp.dot(k, wku1[0:C, :], preferred_element_type=jnp.float32)
    kb_o[...] = jnp.dot(k, wku1[C:2 * C, :], preferred_element_type=jnp.float32)
    va_o[...] = jnp.dot(v, wm1[0:C, :], preferred_element_type=jnp.float32)
    vb_o[...] = jnp.dot(v, wm1[C:2 * C, :], preferred_element_type=jnp.float32)


def _node_tables(x, p):
    Bn = 2000
    full = lambda shape: pl.BlockSpec(shape, lambda i: (0,) * len(shape))
    return pl.pallas_call(
        _node_tables_body,
        grid=(N // Bn,),
        in_specs=[
            pl.BlockSpec((Bn, D), lambda i: (i, 0)),
            full((D, C)), full((1, C)),
            full((D, C)), full((1, C)),
            full((D, C)), full((1, C)),
            full((3 * C, C)), full((3 * C, C)),
        ],
        out_specs=[pl.BlockSpec((Bn, C), lambda i: (i, 0))] * 5,
        out_shape=[jax.ShapeDtypeStruct((N, C), jnp.float32)] * 5,
    )(x, p['Wq'], p['bq'].reshape(1, C), p['Wk'], p['bk'].reshape(1, C),
      p['Wv'], p['bv'].reshape(1, C), p['Wku1'], p['Wm1'])


def _sc_gather_body(dst_h, src_h, q_h, ka_h, kb_h, va_h, vb_h,
                    gq_h, gka_h, gkb_h, gva_h, gvb_h,
                    di, si, bq, bka, bkb, bva, bvb, gsem, osem):
    wid = lax.axis_index("s") * NC + lax.axis_index("c")
    base = wid * EPW

    def chunk(c, carry):
        off = base + c * GB
        pltpu.sync_copy(dst_h.at[pl.ds(off, GB)], di)
        pltpu.sync_copy(src_h.at[pl.ds(off, GB)], si)
        cps = [
            pltpu.async_copy(q_h.at[di], bq, gsem),
            pltpu.async_copy(ka_h.at[di], bka, gsem),
            pltpu.async_copy(kb_h.at[si], bkb, gsem),
            pltpu.async_copy(va_h.at[di], bva, gsem),
            pltpu.async_copy(vb_h.at[si], bvb, gsem),
        ]
        for cp in cps:
            cp.wait()
        ocs = [
            pltpu.async_copy(bq, gq_h.at[pl.ds(off, GB)], osem),
            pltpu.async_copy(bka, gka_h.at[pl.ds(off, GB)], osem),
            pltpu.async_copy(bkb, gkb_h.at[pl.ds(off, GB)], osem),
            pltpu.async_copy(bva, gva_h.at[pl.ds(off, GB)], osem),
            pltpu.async_copy(bvb, gvb_h.at[pl.ds(off, GB)], osem),
        ]
        for oc in ocs:
            oc.wait()
        return carry

    lax.fori_loop(0, EPW // GB, chunk, 0)


def _sc_gather(dst, src, q, ka, kb, va, vb):
    mesh = plsc.VectorSubcoreMesh(core_axis_name="c", subcore_axis_name="s",
                                  num_cores=NC, num_subcores=NS)
    fn = pl.kernel(
        _sc_gather_body,
        out_type=[jax.ShapeDtypeStruct((E, C), jnp.float32)] * 5,
        mesh=mesh,
        scratch_types=[
            pltpu.VMEM((GB,), jnp.int32),
            pltpu.VMEM((GB,), jnp.int32),
            pltpu.VMEM((GB, C), jnp.float32),
            pltpu.VMEM((GB, C), jnp.float32),
            pltpu.VMEM((GB, C), jnp.float32),
            pltpu.VMEM((GB, C), jnp.float32),
            pltpu.VMEM((GB, C), jnp.float32),
            pltpu.SemaphoreType.DMA,
            pltpu.SemaphoreType.DMA,
        ],
    )
    return fn(dst, src, q, ka, kb, va, vb)


def _pass1_body(gka, gq, gkb, ea, wek, ck, wku2, bku2, alpha_o, stats_o):
    i = pl.program_id(0)

    h = gka[...] + gkb[...] + jnp.dot(ea[...], wek[...],
                                      preferred_element_type=jnp.float32) + ck[...]
    h = h * _sig(h)
    kj = jnp.dot(h, wku2[...], preferred_element_type=jnp.float32) + bku2[...]
    alpha = gq[...] * kj * INV_SQRT_C
    alpha_o[...] = alpha

    @pl.when(i == 0)
    def _():
        stats_o[...] = jnp.zeros_like(stats_o)

    s1 = jnp.sum(alpha, axis=0, keepdims=True)
    s2 = jnp.sum(alpha * alpha, axis=0, keepdims=True)
    stats_o[0:1, :] += s1
    stats_o[1:2, :] += s2


def _pass1(gka, gq, gkb, ea, wek, ck, wku2, bku2):
    full = lambda shape: pl.BlockSpec(shape, lambda i: (0,) * len(shape))
    return pl.pallas_call(
        _pass1_body,
        grid=(E // BE,),
        in_specs=[
            pl.BlockSpec((BE, C), lambda i: (i, 0)),
            pl.BlockSpec((BE, C), lambda i: (i, 0)),
            pl.BlockSpec((BE, C), lambda i: (i, 0)),
            pl.BlockSpec((BE, ED), lambda i: (i, 0)),
            full((ED, C)), full((1, C)), full((C, C)), full((1, C)),
        ],
        out_specs=[pl.BlockSpec((BE, C), lambda i: (i, 0)),
                   pl.BlockSpec((8, C), lambda i: (0, 0))],
        out_shape=[jax.ShapeDtypeStruct((E, C), jnp.float32),
                   jax.ShapeDtypeStruct((8, C), jnp.float32)],
    )(gka, gq, gkb, ea, wek, ck, wku2, bku2)


def _pass2_body(alpha, gva, gvb, ea, stats, gatt, batt, wem, cm, wm2, bm2,
                gmsg_o):
    mu = stats[0:1, :] * (1.0 / E)
    ex2 = stats[1:2, :] * (1.0 / E)
    var = ex2 - mu * mu
    inv = gatt[...] * lax.rsqrt(var + BN_EPS)
    shift = batt[...] - mu * inv
    gate = _sig(alpha[...] * inv + shift)

    h = gva[...] + gvb[...] + jnp.dot(ea[...], wem[...],
                                      preferred_element_type=jnp.float32) + cm[...]
    h = h * _sig(h)
    msg = jnp.dot(h, wm2[...], preferred_element_type=jnp.float32) + bm2[...]
    gmsg_o[...] = msg * gate


def _pass2(alpha, gva, gvb, ea, stats, gatt, batt, wem, cm, wm2, bm2):
    full = lambda shape: pl.BlockSpec(shape, lambda i: (0,) * len(shape))
    return pl.pallas_call(
        _pass2_body,
        grid=(E // BE,),
        in_specs=[
            pl.BlockSpec((BE, C), lambda i: (i, 0)),
            pl.BlockSpec((BE, C), lambda i: (i, 0)),
            pl.BlockSpec((BE, C), lambda i: (i, 0)),
            pl.BlockSpec((BE, ED), lambda i: (i, 0)),
            full((8, C)), full((1, C)), full((1, C)),
            full((ED, C)), full((1, C)), full((C, C)), full((1, C)),
        ],
        out_specs=pl.BlockSpec((BE, C), lambda i: (i, 0)),
        out_shape=jax.ShapeDtypeStruct((E, C), jnp.float32),
    )(alpha, gva, gvb, ea, stats, gatt, batt, wem, cm, wm2, bm2)


def _sc_scatter_body(dst_h, gmsg_h, zeros_h, parts_h, idxb, datab, acc):
    cid = lax.axis_index("c")
    sid = lax.axis_index("s")
    wid = sid * NC + cid
    base = wid * EPW
    r0 = sid * INIT_ROWS

    @pl.when(sid < N // INIT_ROWS)
    def _():
        pltpu.sync_copy(zeros_h.at[pl.ds(r0, INIT_ROWS)],
                        acc.at[pl.ds(r0, INIT_ROWS)])
    plsc.subcore_barrier()

    def chunk(c, carry):
        off = base + c * SB
        pltpu.sync_copy(dst_h.at[pl.ds(off, SB)], idxb)
        pltpu.sync_copy(gmsg_h.at[pl.ds(off, SB)], datab)
        pltpu.sync_copy(datab, acc.at[idxb], add=True)
        return carry

    lax.fori_loop(0, EPW // SB, chunk, 0)
    plsc.subcore_barrier()

    @pl.when(sid < N // INIT_ROWS)
    def _():
        pltpu.sync_copy(acc.at[pl.ds(r0, INIT_ROWS)],
                        parts_h.at[cid, pl.ds(r0, INIT_ROWS)])


def _sc_scatter(dst, gmsg, zeros):
    mesh = plsc.VectorSubcoreMesh(core_axis_name="c", subcore_axis_name="s",
                                  num_cores=NC, num_subcores=NS)
    fn = pl.kernel(
        _sc_scatter_body,
        out_type=jax.ShapeDtypeStruct((NC, N, C), jnp.float32),
        mesh=mesh,
        scratch_types=[
            pltpu.VMEM((SB,), jnp.int32),
            pltpu.VMEM((SB, C), jnp.float32),
            pltpu.VMEM_SHARED((N, C), jnp.float32),
        ],
    )
    return fn(dst, gmsg, zeros)


def _final_body(parts, x_ref, wc, bc, gbn, bbn, out_o):
    agg = parts[0] + parts[1]
    o = jnp.dot(agg, wc[...], preferred_element_type=jnp.float32) + bc[...]
    mu = jnp.sum(o, axis=0, keepdims=True) * (1.0 / N)
    ex2 = jnp.sum(o * o, axis=0, keepdims=True) * (1.0 / N)
    var = ex2 - mu * mu
    inv = gbn[...] * lax.rsqrt(var + BN_EPS)
    z = x_ref[...] + o * inv + (bbn[...] - mu * inv)
    out_o[...] = jnp.maximum(z, 0.0) + jnp.log(1.0 + jnp.exp(-jnp.abs(z)))


def _final(parts, x, wc, bc, gbn, bbn):
    return pl.pallas_call(
        _final_body,
        out_shape=jax.ShapeDtypeStruct((N, C), jnp.float32),
    )(parts, x, wc, bc.reshape(1, C), gbn.reshape(1, C), bbn.reshape(1, C))


def kernel(x, edge_index, edge_attr, params):
    p = params
    ei = edge_index.astype(jnp.int32)
    src = ei[0]
    dst = ei[1]

    wek = p['We'] @ p['Wku1'][2 * C:]
    ck = (p['bku1'] + p['be'] @ p['Wku1'][2 * C:]).reshape(1, C)
    wem = p['We'] @ p['Wm1'][2 * C:]
    cm = (p['bm1'] + p['be'] @ p['Wm1'][2 * C:]).reshape(1, C)

    q, ka, kb, va, vb = _node_tables(x, p)
    gq, gka, gkb, gva, gvb = _sc_gather(dst, src, q, ka, kb, va, vb)
    alpha, stats = _pass1(gka, gq, gkb, edge_attr, wek, ck, p['Wku2'],
                          p['bku2'].reshape(1, C))
    gmsg = _pass2(alpha, gva, gvb, edge_attr, stats,
                  p['g_att'].reshape(1, C), p['b_att'].reshape(1, C),
                  wem, cm, p['Wm2'], p['bm2'].reshape(1, C))
    zeros = jnp.zeros((N, C), jnp.float32)
    parts = _sc_scatter(dst, gmsg, zeros)
    return _final(parts, x, p['Wc'], p['bc'], p['g_bn'], p['b_bn'])


# speedup vs baseline: 3.0751x; 1.0502x over previous
import functools

import jax
import jax.numpy as jnp
import numpy as np
from jax import lax
from jax.experimental import pallas as pl
from jax.experimental.pallas import tpu as pltpu
from jax.experimental.pallas import tpu_sc as plsc

N = 10000
E = 320000
D = 128
ED = 16
C = 128

NC = 2
NS = 16
NW = NC * NS
EPW = E // NW
GB = 80
SB = 200
INIT_ROWS = 1000

BE = 1600
INV_SQRT_C = 1.0 / float(np.sqrt(C))
BN_EPS = 1e-5


def _sig(z):
    return 1.0 / (1.0 + jnp.exp(-z))


def _node_tables_body(x_ref, wq, bq, wk, bk, wv, bv, wku1, wm1,
                      q_o, ka_o, kb_o, va_o, vb_o):
    xb = x_ref[...]
    q = jnp.dot(xb, wq[...], preferred_element_type=jnp.float32) + bq[...]
    k = jnp.dot(xb, wk[...], preferred_element_type=jnp.float32) + bk[...]
    v = jnp.dot(xb, wv[...], preferred_element_type=jnp.float32) + bv[...]
    q_o[...] = q
    ka_o[...] = jnp.dot(k, wku1[0:C, :], preferred_element_type=jnp.float32)
    kb_o[...] = jnp.dot(k, wku1[C:2 * C, :], preferred_element_type=jnp.float32)
    va_o[...] = jnp.dot(v, wm1[0:C, :], preferred_element_type=jnp.float32)
    vb_o[...] = jnp.dot(v, wm1[C:2 * C, :], preferred_element_type=jnp.float32)


def _node_tables(x, p):
    Bn = 2000
    full = lambda shape: pl.BlockSpec(shape, lambda i: (0,) * len(shape))
    return pl.pallas_call(
        _node_tables_body,
        grid=(N // Bn,),
        in_specs=[
            pl.BlockSpec((Bn, D), lambda i: (i, 0)),
            full((D, C)), full((1, C)),
            full((D, C)), full((1, C)),
            full((D, C)), full((1, C)),
            full((3 * C, C)), full((3 * C, C)),
        ],
        out_specs=[pl.BlockSpec((Bn, C), lambda i: (i, 0))] * 5,
        out_shape=[jax.ShapeDtypeStruct((N, C), jnp.float32)] * 5,
    )(x, p['Wq'], p['bq'].reshape(1, C), p['Wk'], p['bk'].reshape(1, C),
      p['Wv'], p['bv'].reshape(1, C), p['Wku1'], p['Wm1'])


def _sc_gather_body(dst_h, src_h, q_h, ka_h, kb_h, va_h, vb_h,
                    gq_h, ghk_h, ghm_h,
                    di, si, bq, bhk, bhm, gsem, osem):
    wid = lax.axis_index("s") * NC + lax.axis_index("c")
    base = wid * EPW

    def chunk(c, carry):
        off = base + c * GB
        pltpu.sync_copy(dst_h.at[pl.ds(off, GB)], di)
        pltpu.sync_copy(src_h.at[pl.ds(off, GB)], si)
        cps = [
            pltpu.async_copy(q_h.at[di], bq, gsem),
            pltpu.async_copy(ka_h.at[di], bhk, gsem),
            pltpu.async_copy(va_h.at[di], bhm, gsem),
        ]
        for cp in cps:
            cp.wait()
        cps2 = [
            pltpu.async_copy(kb_h.at[si], bhk, gsem, add=True),
            pltpu.async_copy(vb_h.at[si], bhm, gsem, add=True),
        ]
        for cp in cps2:
            cp.wait()
        ocs = [
            pltpu.async_copy(bq, gq_h.at[pl.ds(off, GB)], osem),
            pltpu.async_copy(bhk, ghk_h.at[pl.ds(off, GB)], osem),
            pltpu.async_copy(bhm, ghm_h.at[pl.ds(off, GB)], osem),
        ]
        for oc in ocs:
            oc.wait()
        return carry

    lax.fori_loop(0, EPW // GB, chunk, 0)


def _sc_gather(dst, src, q, ka, kb, va, vb):
    mesh = plsc.VectorSubcoreMesh(core_axis_name="c", subcore_axis_name="s",
                                  num_cores=NC, num_subcores=NS)
    fn = pl.kernel(
        _sc_gather_body,
        out_type=[jax.ShapeDtypeStruct((E, C), jnp.float32)] * 3,
        mesh=mesh,
        scratch_types=[
            pltpu.VMEM((GB,), jnp.int32),
            pltpu.VMEM((GB,), jnp.int32),
            pltpu.VMEM((GB, C), jnp.float32),
            pltpu.VMEM((GB, C), jnp.float32),
            pltpu.VMEM((GB, C), jnp.float32),
            pltpu.SemaphoreType.DMA,
            pltpu.SemaphoreType.DMA,
        ],
    )
    return fn(dst, src, q, ka, kb, va, vb)


def _pass1_body(ghk, gq, ea, wek, ck, wku2, bku2, alpha_o, stats_o):
    i = pl.program_id(0)

    h = ghk[...] + jnp.dot(ea[...], wek[...],
                           preferred_element_type=jnp.float32) + ck[...]
    h = h * _sig(h)
    kj = jnp.dot(h, wku2[...], preferred_element_type=jnp.float32) + bku2[...]
    alpha = gq[...] * kj * INV_SQRT_C
    alpha_o[...] = alpha

    @pl.when(i == 0)
    def _():
        stats_o[...] = jnp.zeros_like(stats_o)

    s1 = jnp.sum(alpha, axis=0, keepdims=True)
    s2 = jnp.sum(alpha * alpha, axis=0, keepdims=True)
    stats_o[0:1, :] += s1
    stats_o[1:2, :] += s2


def _pass1(ghk, gq, ea, wek, ck, wku2, bku2):
    full = lambda shape: pl.BlockSpec(shape, lambda i: (0,) * len(shape))
    return pl.pallas_call(
        _pass1_body,
        grid=(E // BE,),
        in_specs=[
            pl.BlockSpec((BE, C), lambda i: (i, 0)),
            pl.BlockSpec((BE, C), lambda i: (i, 0)),
            pl.BlockSpec((BE, ED), lambda i: (i, 0)),
            full((ED, C)), full((1, C)), full((C, C)), full((1, C)),
        ],
        out_specs=[pl.BlockSpec((BE, C), lambda i: (i, 0)),
                   pl.BlockSpec((8, C), lambda i: (0, 0))],
        out_shape=[jax.ShapeDtypeStruct((E, C), jnp.float32),
                   jax.ShapeDtypeStruct((8, C), jnp.float32)],
    )(ghk, gq, ea, wek, ck, wku2, bku2)


def _pass2_body(alpha, ghm, ea, stats, gatt, batt, wem, cm, wm2, bm2,
                gmsg_o):
    mu = stats[0:1, :] * (1.0 / E)
    ex2 = stats[1:2, :] * (1.0 / E)
    var = ex2 - mu * mu
    inv = gatt[...] * lax.rsqrt(var + BN_EPS)
    shift = batt[...] - mu * inv
    gate = _sig(alpha[...] * inv + shift)

    h = ghm[...] + jnp.dot(ea[...], wem[...],
                           preferred_element_type=jnp.float32) + cm[...]
    h = h * _sig(h)
    msg = jnp.dot(h, wm2[...], preferred_element_type=jnp.float32) + bm2[...]
    gmsg_o[...] = msg * gate


def _pass2(alpha, ghm, ea, stats, gatt, batt, wem, cm, wm2, bm2):
    full = lambda shape: pl.BlockSpec(shape, lambda i: (0,) * len(shape))
    return pl.pallas_call(
        _pass2_body,
        grid=(E // BE,),
        in_specs=[
            pl.BlockSpec((BE, C), lambda i: (i, 0)),
            pl.BlockSpec((BE, C), lambda i: (i, 0)),
            pl.BlockSpec((BE, ED), lambda i: (i, 0)),
            full((8, C)), full((1, C)), full((1, C)),
            full((ED, C)), full((1, C)), full((C, C)), full((1, C)),
        ],
        out_specs=pl.BlockSpec((BE, C), lambda i: (i, 0)),
        out_shape=jax.ShapeDtypeStruct((E, C), jnp.float32),
    )(alpha, ghm, ea, stats, gatt, batt, wem, cm, wm2, bm2)


def _sc_scatter_body(dst_h, gmsg_h, zeros_h, parts_h, idxb, datab, acc):
    cid = lax.axis_index("c")
    sid = lax.axis_index("s")
    wid = sid * NC + cid
    base = wid * EPW
    r0 = sid * INIT_ROWS

    @pl.when(sid < N // INIT_ROWS)
    def _():
        pltpu.sync_copy(zeros_h.at[pl.ds(r0, INIT_ROWS)],
                        acc.at[pl.ds(r0, INIT_ROWS)])
    plsc.subcore_barrier()

    def chunk(c, carry):
        off = base + c * SB
        pltpu.sync_copy(dst_h.at[pl.ds(off, SB)], idxb)
        pltpu.sync_copy(gmsg_h.at[pl.ds(off, SB)], datab)
        pltpu.sync_copy(datab, acc.at[idxb], add=True)
        return carry

    lax.fori_loop(0, EPW // SB, chunk, 0)
    plsc.subcore_barrier()

    @pl.when(sid < N // INIT_ROWS)
    def _():
        pltpu.sync_copy(acc.at[pl.ds(r0, INIT_ROWS)],
                        parts_h.at[cid, pl.ds(r0, INIT_ROWS)])


def _sc_scatter(dst, gmsg, zeros):
    mesh = plsc.VectorSubcoreMesh(core_axis_name="c", subcore_axis_name="s",
                                  num_cores=NC, num_subcores=NS)
    fn = pl.kernel(
        _sc_scatter_body,
        out_type=jax.ShapeDtypeStruct((NC, N, C), jnp.float32),
        mesh=mesh,
        scratch_types=[
            pltpu.VMEM((SB,), jnp.int32),
            pltpu.VMEM((SB, C), jnp.float32),
            pltpu.VMEM_SHARED((N, C), jnp.float32),
        ],
    )
    return fn(dst, gmsg, zeros)


def _final_body(parts, x_ref, wc, bc, gbn, bbn, out_o):
    agg = parts[0] + parts[1]
    o = jnp.dot(agg, wc[...], preferred_element_type=jnp.float32) + bc[...]
    mu = jnp.sum(o, axis=0, keepdims=True) * (1.0 / N)
    ex2 = jnp.sum(o * o, axis=0, keepdims=True) * (1.0 / N)
    var = ex2 - mu * mu
    inv = gbn[...] * lax.rsqrt(var + BN_EPS)
    z = x_ref[...] + o * inv + (bbn[...] - mu * inv)
    out_o[...] = jnp.maximum(z, 0.0) + jnp.log(1.0 + jnp.exp(-jnp.abs(z)))


def _final(parts, x, wc, bc, gbn, bbn):
    return pl.pallas_call(
        _final_body,
        out_shape=jax.ShapeDtypeStruct((N, C), jnp.float32),
    )(parts, x, wc, bc.reshape(1, C), gbn.reshape(1, C), bbn.reshape(1, C))


def kernel(x, edge_index, edge_attr, params):
    p = params
    ei = edge_index.astype(jnp.int32)
    src = ei[0]
    dst = ei[1]

    wek = p['We'] @ p['Wku1'][2 * C:]
    ck = (p['bku1'] + p['be'] @ p['Wku1'][2 * C:]).reshape(1, C)
    wem = p['We'] @ p['Wm1'][2 * C:]
    cm = (p['bm1'] + p['be'] @ p['Wm1'][2 * C:]).reshape(1, C)

    q, ka, kb, va, vb = _node_tables(x, p)
    gq, ghk, ghm = _sc_gather(dst, src, q, ka, kb, va, vb)
    alpha, stats = _pass1(ghk, gq, edge_attr, wek, ck, p['Wku2'],
                          p['bku2'].reshape(1, C))
    gmsg = _pass2(alpha, ghm, edge_attr, stats,
                  p['g_att'].reshape(1, C), p['b_att'].reshape(1, C),
                  wem, cm, p['Wm2'], p['bm2'].reshape(1, C))
    zeros = jnp.zeros((N, C), jnp.float32)
    parts = _sc_scatter(dst, gmsg, zeros)
    return _final(parts, x, p['Wc'], p['bc'], p['g_bn'], p['b_bn'])


# speedup vs baseline: 4.0468x; 1.3160x over previous
import functools

import jax
import jax.numpy as jnp
import numpy as np
from jax import lax
from jax.experimental import pallas as pl
from jax.experimental.pallas import tpu as pltpu
from jax.experimental.pallas import tpu_sc as plsc

N = 10000
E = 320000
D = 128
ED = 16
C = 128

NC = 2
NS = 16
NW = NC * NS
EPW = E // NW
GB = 200
SB = 200
INIT_ROWS = 1000

BE = 4000
INV_SQRT_C = 1.0 / float(np.sqrt(C))
BN_EPS = 1e-5


def _sig(z):
    return 1.0 / (1.0 + jnp.exp(-z))


def _node_tables_body(x_ref, wq, bq, wk, bk, wv, bv, wku1, wm1,
                      q_o, ka_o, kb_o, va_o, vb_o):
    xb = x_ref[...]
    q = jnp.dot(xb, wq[...], preferred_element_type=jnp.float32) + bq[...]
    k = jnp.dot(xb, wk[...], preferred_element_type=jnp.float32) + bk[...]
    v = jnp.dot(xb, wv[...], preferred_element_type=jnp.float32) + bv[...]
    q_o[...] = q
    ka_o[...] = jnp.dot(k, wku1[0:C, :], preferred_element_type=jnp.float32)
    kb_o[...] = jnp.dot(k, wku1[C:2 * C, :], preferred_element_type=jnp.float32)
    va_o[...] = jnp.dot(v, wm1[0:C, :], preferred_element_type=jnp.float32)
    vb_o[...] = jnp.dot(v, wm1[C:2 * C, :], preferred_element_type=jnp.float32)


def _node_tables(x, p):
    Bn = 2000
    full = lambda shape: pl.BlockSpec(shape, lambda i: (0,) * len(shape))
    return pl.pallas_call(
        _node_tables_body,
        grid=(N // Bn,),
        in_specs=[
            pl.BlockSpec((Bn, D), lambda i: (i, 0)),
            full((D, C)), full((1, C)),
            full((D, C)), full((1, C)),
            full((D, C)), full((1, C)),
            full((3 * C, C)), full((3 * C, C)),
        ],
        out_specs=[pl.BlockSpec((Bn, C), lambda i: (i, 0))] * 5,
        out_shape=[jax.ShapeDtypeStruct((N, C), jnp.float32)] * 5,
    )(x, p['Wq'], p['bq'].reshape(1, C), p['Wk'], p['bk'].reshape(1, C),
      p['Wv'], p['bv'].reshape(1, C), p['Wku1'], p['Wm1'])


def _gather_k_body(dst_h, src_h, q_h, ka_h, kb_h, gq_h, ghk_h,
                   di0, si0, bq0, bhk0, qsem0, ksem0, osem0,
                   di1, si1, bq1, bhk1, qsem1, ksem1, osem1):
    wid = lax.axis_index("s") * NC + lax.axis_index("c")
    base = wid * EPW
    sets = ((di0, si0, bq0, bhk0, qsem0, ksem0, osem0),
            (di1, si1, bq1, bhk1, qsem1, ksem1, osem1))

    def pair(p, carry):
        for s in (0, 1):
            di, si, bq, bhk, qsem, ksem, osem = sets[s]
            c = 2 * p + s
            off = base + c * GB

            @pl.when(c >= 2)
            def _():
                pltpu.make_async_copy(bq, gq_h.at[pl.ds(off, GB)], osem).wait()
                pltpu.make_async_copy(bhk, ghk_h.at[pl.ds(off, GB)], osem).wait()

            pltpu.sync_copy(dst_h.at[pl.ds(off, GB)], di)
            pltpu.sync_copy(src_h.at[pl.ds(off, GB)], si)
            gq_cp = pltpu.async_copy(q_h.at[di], bq, qsem)
            ka_cp = pltpu.async_copy(ka_h.at[di], bhk, ksem)
            ka_cp.wait()
            kb_cp = pltpu.async_copy(kb_h.at[si], bhk, ksem, add=True)
            gq_cp.wait()
            pltpu.async_copy(bq, gq_h.at[pl.ds(off, GB)], osem)
            kb_cp.wait()
            pltpu.async_copy(bhk, ghk_h.at[pl.ds(off, GB)], osem)
        return carry

    lax.fori_loop(0, EPW // GB // 2, pair, 0)
    for s in (0, 1):
        di, si, bq, bhk, qsem, ksem, osem = sets[s]
        pltpu.make_async_copy(bq, gq_h.at[pl.ds(base, GB)], osem).wait()
        pltpu.make_async_copy(bhk, ghk_h.at[pl.ds(base, GB)], osem).wait()


def _gather_m_body(dst_h, src_h, va_h, vb_h, ghm_h,
                   di0, si0, bhm0, gsem0, osem0,
                   di1, si1, bhm1, gsem1, osem1):
    wid = lax.axis_index("s") * NC + lax.axis_index("c")
    base = wid * EPW
    sets = ((di0, si0, bhm0, gsem0, osem0),
            (di1, si1, bhm1, gsem1, osem1))

    def pair(p, carry):
        for s in (0, 1):
            di, si, bhm, gsem, osem = sets[s]
            c = 2 * p + s
            off = base + c * GB

            @pl.when(c >= 2)
            def _():
                pltpu.make_async_copy(bhm, ghm_h.at[pl.ds(off, GB)], osem).wait()

            pltpu.sync_copy(dst_h.at[pl.ds(off, GB)], di)
            pltpu.sync_copy(src_h.at[pl.ds(off, GB)], si)
            pltpu.async_copy(va_h.at[di], bhm, gsem).wait()
            pltpu.async_copy(vb_h.at[si], bhm, gsem, add=True).wait()
            pltpu.async_copy(bhm, ghm_h.at[pl.ds(off, GB)], osem)
        return carry

    lax.fori_loop(0, EPW // GB // 2, pair, 0)
    for s in (0, 1):
        di, si, bhm, gsem, osem = sets[s]
        pltpu.make_async_copy(bhm, ghm_h.at[pl.ds(base, GB)], osem).wait()


def _sc_gather_k(dst, src, q, ka, kb):
    mesh = plsc.VectorSubcoreMesh(core_axis_name="c", subcore_axis_name="s",
                                  num_cores=NC, num_subcores=NS)
    set_scratch = [
        pltpu.VMEM((GB,), jnp.int32),
        pltpu.VMEM((GB,), jnp.int32),
        pltpu.VMEM((GB, C), jnp.float32),
        pltpu.VMEM((GB, C), jnp.float32),
        pltpu.SemaphoreType.DMA,
        pltpu.SemaphoreType.DMA,
        pltpu.SemaphoreType.DMA,
    ]
    fn = pl.kernel(
        _gather_k_body,
        out_type=[jax.ShapeDtypeStruct((E, C), jnp.float32)] * 2,
        mesh=mesh,
        scratch_types=set_scratch + set_scratch,
    )
    return fn(dst, src, q, ka, kb)


def _sc_gather_m(dst, src, va, vb):
    mesh = plsc.VectorSubcoreMesh(core_axis_name="c", subcore_axis_name="s",
                                  num_cores=NC, num_subcores=NS)
    set_scratch = [
        pltpu.VMEM((GB,), jnp.int32),
        pltpu.VMEM((GB,), jnp.int32),
        pltpu.VMEM((GB, C), jnp.float32),
        pltpu.SemaphoreType.DMA,
        pltpu.SemaphoreType.DMA,
    ]
    fn = pl.kernel(
        _gather_m_body,
        out_type=jax.ShapeDtypeStruct((E, C), jnp.float32),
        mesh=mesh,
        scratch_types=set_scratch + set_scratch,
    )
    return fn(dst, src, va, vb)


def _pass1_body(ghk, gq, ea, wek, ck, wku2, bku2, alpha_o, stats_o):
    i = pl.program_id(0)

    h = ghk[...] + jnp.dot(ea[...], wek[...],
                           preferred_element_type=jnp.float32) + ck[...]
    h = h * _sig(h)
    kj = jnp.dot(h, wku2[...], preferred_element_type=jnp.float32) + bku2[...]
    alpha = gq[...] * kj * INV_SQRT_C
    alpha_o[...] = alpha

    @pl.when(i == 0)
    def _():
        stats_o[...] = jnp.zeros_like(stats_o)

    s1 = jnp.sum(alpha, axis=0, keepdims=True)
    s2 = jnp.sum(alpha * alpha, axis=0, keepdims=True)
    stats_o[0:1, :] += s1
    stats_o[1:2, :] += s2


def _pass1(ghk, gq, ea, wek, ck, wku2, bku2):
    full = lambda shape: pl.BlockSpec(shape, lambda i: (0,) * len(shape))
    return pl.pallas_call(
        _pass1_body,
        grid=(E // BE,),
        in_specs=[
            pl.BlockSpec((BE, C), lambda i: (i, 0)),
            pl.BlockSpec((BE, C), lambda i: (i, 0)),
            pl.BlockSpec((BE, ED), lambda i: (i, 0)),
            full((ED, C)), full((1, C)), full((C, C)), full((1, C)),
        ],
        out_specs=[pl.BlockSpec((BE, C), lambda i: (i, 0)),
                   pl.BlockSpec((8, C), lambda i: (0, 0))],
        out_shape=[jax.ShapeDtypeStruct((E, C), jnp.float32),
                   jax.ShapeDtypeStruct((8, C), jnp.float32)],
    )(ghk, gq, ea, wek, ck, wku2, bku2)


def _pass2_body(alpha, ghm, ea, stats, gatt, batt, wem, cm, wm2, bm2,
                gmsg_o):
    mu = stats[0:1, :] * (1.0 / E)
    ex2 = stats[1:2, :] * (1.0 / E)
    var = ex2 - mu * mu
    inv = gatt[...] * lax.rsqrt(var + BN_EPS)
    shift = batt[...] - mu * inv
    gate = _sig(alpha[...] * inv + shift)

    h = ghm[...] + jnp.dot(ea[...], wem[...],
                           preferred_element_type=jnp.float32) + cm[...]
    h = h * _sig(h)
    msg = jnp.dot(h, wm2[...], preferred_element_type=jnp.float32) + bm2[...]
    gmsg_o[...] = msg * gate


def _pass2(alpha, ghm, ea, stats, gatt, batt, wem, cm, wm2, bm2):
    full = lambda shape: pl.BlockSpec(shape, lambda i: (0,) * len(shape))
    return pl.pallas_call(
        _pass2_body,
        grid=(E // BE,),
        in_specs=[
            pl.BlockSpec((BE, C), lambda i: (i, 0)),
            pl.BlockSpec((BE, C), lambda i: (i, 0)),
            pl.BlockSpec((BE, ED), lambda i: (i, 0)),
            full((8, C)), full((1, C)), full((1, C)),
            full((ED, C)), full((1, C)), full((C, C)), full((1, C)),
        ],
        out_specs=pl.BlockSpec((BE, C), lambda i: (i, 0)),
        out_shape=jax.ShapeDtypeStruct((E, C), jnp.float32),
    )(alpha, ghm, ea, stats, gatt, batt, wem, cm, wm2, bm2)


def _sc_scatter_body(dst_h, gmsg_h, zeros_h, parts_h, idxb, datab, acc):
    cid = lax.axis_index("c")
    sid = lax.axis_index("s")
    wid = sid * NC + cid
    base = wid * EPW
    r0 = sid * INIT_ROWS

    @pl.when(sid < N // INIT_ROWS)
    def _():
        pltpu.sync_copy(zeros_h.at[pl.ds(r0, INIT_ROWS)],
                        acc.at[pl.ds(r0, INIT_ROWS)])
    plsc.subcore_barrier()

    def chunk(c, carry):
        off = base + c * SB
        pltpu.sync_copy(dst_h.at[pl.ds(off, SB)], idxb)
        pltpu.sync_copy(gmsg_h.at[pl.ds(off, SB)], datab)
        pltpu.sync_copy(datab, acc.at[idxb], add=True)
        return carry

    lax.fori_loop(0, EPW // SB, chunk, 0)
    plsc.subcore_barrier()

    @pl.when(sid < N // INIT_ROWS)
    def _():
        pltpu.sync_copy(acc.at[pl.ds(r0, INIT_ROWS)],
                        parts_h.at[cid, pl.ds(r0, INIT_ROWS)])


def _sc_scatter(dst, gmsg, zeros):
    mesh = plsc.VectorSubcoreMesh(core_axis_name="c", subcore_axis_name="s",
                                  num_cores=NC, num_subcores=NS)
    fn = pl.kernel(
        _sc_scatter_body,
        out_type=jax.ShapeDtypeStruct((NC, N, C), jnp.float32),
        mesh=mesh,
        scratch_types=[
            pltpu.VMEM((SB,), jnp.int32),
            pltpu.VMEM((SB, C), jnp.float32),
            pltpu.VMEM_SHARED((N, C), jnp.float32),
        ],
    )
    return fn(dst, gmsg, zeros)


def _final_body(parts, x_ref, wc, bc, gbn, bbn, out_o):
    agg = parts[0] + parts[1]
    o = jnp.dot(agg, wc[...], preferred_element_type=jnp.float32) + bc[...]
    mu = jnp.sum(o, axis=0, keepdims=True) * (1.0 / N)
    ex2 = jnp.sum(o * o, axis=0, keepdims=True) * (1.0 / N)
    var = ex2 - mu * mu
    inv = gbn[...] * lax.rsqrt(var + BN_EPS)
    z = x_ref[...] + o * inv + (bbn[...] - mu * inv)
    out_o[...] = jnp.maximum(z, 0.0) + jnp.log(1.0 + jnp.exp(-jnp.abs(z)))


def _final(parts, x, wc, bc, gbn, bbn):
    return pl.pallas_call(
        _final_body,
        out_shape=jax.ShapeDtypeStruct((N, C), jnp.float32),
    )(parts, x, wc, bc.reshape(1, C), gbn.reshape(1, C), bbn.reshape(1, C))


def kernel(x, edge_index, edge_attr, params):
    p = params
    ei = edge_index.astype(jnp.int32)
    src = ei[0]
    dst = ei[1]

    wek = p['We'] @ p['Wku1'][2 * C:]
    ck = (p['bku1'] + p['be'] @ p['Wku1'][2 * C:]).reshape(1, C)
    wem = p['We'] @ p['Wm1'][2 * C:]
    cm = (p['bm1'] + p['be'] @ p['Wm1'][2 * C:]).reshape(1, C)

    q, ka, kb, va, vb = _node_tables(x, p)
    gq, ghk = _sc_gather_k(dst, src, q, ka, kb)
    ghm = _sc_gather_m(dst, src, va, vb)
    alpha, stats = _pass1(ghk, gq, edge_attr, wek, ck, p['Wku2'],
                          p['bku2'].reshape(1, C))
    gmsg = _pass2(alpha, ghm, edge_attr, stats,
                  p['g_att'].reshape(1, C), p['b_att'].reshape(1, C),
                  wem, cm, p['Wm2'], p['bm2'].reshape(1, C))
    zeros = jnp.zeros((N, C), jnp.float32)
    parts = _sc_scatter(dst, gmsg, zeros)
    return _final(parts, x, p['Wc'], p['bc'], p['g_bn'], p['b_bn'])


# speedup vs baseline: 4.2876x; 1.0595x over previous
import functools

import jax
import jax.numpy as jnp
import numpy as np
from jax import lax
from jax.experimental import pallas as pl
from jax.experimental.pallas import tpu as pltpu
from jax.experimental.pallas import tpu_sc as plsc

N = 10000
E = 320000
D = 128
ED = 16
C = 128

NC = 2
NS = 16
NW = NC * NS
EPW = E // NW
GB = 200
SB = 200
INIT_ROWS = 1000

BE = 3200
INV_SQRT_C = 1.0 / float(np.sqrt(C))
BN_EPS = 1e-5


def _sig(z):
    return 1.0 / (1.0 + jnp.exp(-z))


def _node_tables_body(x_ref, wq, bq, wk, bk, wv, bv, wku1, wm1,
                      q_o, ka_o, kb_o, va_o, vb_o):
    xb = x_ref[...]
    q = jnp.dot(xb, wq[...], preferred_element_type=jnp.float32) + bq[...]
    k = jnp.dot(xb, wk[...], preferred_element_type=jnp.float32) + bk[...]
    v = jnp.dot(xb, wv[...], preferred_element_type=jnp.float32) + bv[...]
    q_o[...] = q
    ka_o[...] = jnp.dot(k, wku1[0:C, :], preferred_element_type=jnp.float32)
    kb_o[...] = jnp.dot(k, wku1[C:2 * C, :], preferred_element_type=jnp.float32)
    va_o[...] = jnp.dot(v, wm1[0:C, :], preferred_element_type=jnp.float32)
    vb_o[...] = jnp.dot(v, wm1[C:2 * C, :], preferred_element_type=jnp.float32)


def _node_tables(x, p):
    Bn = 2000
    full = lambda shape: pl.BlockSpec(shape, lambda i: (0,) * len(shape))
    return pl.pallas_call(
        _node_tables_body,
        grid=(N // Bn,),
        in_specs=[
            pl.BlockSpec((Bn, D), lambda i: (i, 0)),
            full((D, C)), full((1, C)),
            full((D, C)), full((1, C)),
            full((D, C)), full((1, C)),
            full((3 * C, C)), full((3 * C, C)),
        ],
        out_specs=[pl.BlockSpec((Bn, C), lambda i: (i, 0))] * 5,
        out_shape=[jax.ShapeDtypeStruct((N, C), jnp.float32)] * 5,
    )(x, p['Wq'], p['bq'].reshape(1, C), p['Wk'], p['bk'].reshape(1, C),
      p['Wv'], p['bv'].reshape(1, C), p['Wku1'], p['Wm1'])


def _gather_k_body(dst_h, src_h, q_h, ka_h, kb_h, gq_h, ghk_h,
                   di0, si0, bq0, bhk0, qsem0, ksem0, osem0,
                   di1, si1, bq1, bhk1, qsem1, ksem1, osem1):
    wid = lax.axis_index("s") * NC + lax.axis_index("c")
    base = wid * EPW
    sets = ((di0, si0, bq0, bhk0, qsem0, ksem0, osem0),
            (di1, si1, bq1, bhk1, qsem1, ksem1, osem1))

    def pair(p, carry):
        for s in (0, 1):
            di, si, bq, bhk, qsem, ksem, osem = sets[s]
            c = 2 * p + s
            off = base + c * GB

            @pl.when(c >= 2)
            def _():
                pltpu.make_async_copy(bq, gq_h.at[pl.ds(off, GB)], osem).wait()
                pltpu.make_async_copy(bhk, ghk_h.at[pl.ds(off, GB)], osem).wait()

            pltpu.sync_copy(dst_h.at[pl.ds(off, GB)], di)
            pltpu.sync_copy(src_h.at[pl.ds(off, GB)], si)
            gq_cp = pltpu.async_copy(q_h.at[di], bq, qsem)
            ka_cp = pltpu.async_copy(ka_h.at[di], bhk, ksem)
            ka_cp.wait()
            kb_cp = pltpu.async_copy(kb_h.at[si], bhk, ksem, add=True)
            gq_cp.wait()
            pltpu.async_copy(bq, gq_h.at[pl.ds(off, GB)], osem)
            kb_cp.wait()
            pltpu.async_copy(bhk, ghk_h.at[pl.ds(off, GB)], osem)
        return carry

    lax.fori_loop(0, EPW // GB // 2, pair, 0)
    for s in (0, 1):
        di, si, bq, bhk, qsem, ksem, osem = sets[s]
        pltpu.make_async_copy(bq, gq_h.at[pl.ds(base, GB)], osem).wait()
        pltpu.make_async_copy(bhk, ghk_h.at[pl.ds(base, GB)], osem).wait()


def _gather_m_body(dst_h, src_h, va_h, vb_h, ghm_h,
                   di0, si0, bhm0, gsem0, osem0,
                   di1, si1, bhm1, gsem1, osem1):
    wid = lax.axis_index("s") * NC + lax.axis_index("c")
    base = wid * EPW
    sets = ((di0, si0, bhm0, gsem0, osem0),
            (di1, si1, bhm1, gsem1, osem1))

    def pair(p, carry):
        for s in (0, 1):
            di, si, bhm, gsem, osem = sets[s]
            c = 2 * p + s
            off = base + c * GB

            @pl.when(c >= 2)
            def _():
                pltpu.make_async_copy(bhm, ghm_h.at[pl.ds(off, GB)], osem).wait()

            pltpu.sync_copy(dst_h.at[pl.ds(off, GB)], di)
            pltpu.sync_copy(src_h.at[pl.ds(off, GB)], si)
            pltpu.async_copy(va_h.at[di], bhm, gsem).wait()
            pltpu.async_copy(vb_h.at[si], bhm, gsem, add=True).wait()
            pltpu.async_copy(bhm, ghm_h.at[pl.ds(off, GB)], osem)
        return carry

    lax.fori_loop(0, EPW // GB // 2, pair, 0)
    for s in (0, 1):
        di, si, bhm, gsem, osem = sets[s]
        pltpu.make_async_copy(bhm, ghm_h.at[pl.ds(base, GB)], osem).wait()


def _sc_gather_k(dst, src, q, ka, kb):
    mesh = plsc.VectorSubcoreMesh(core_axis_name="c", subcore_axis_name="s",
                                  num_cores=NC, num_subcores=NS)
    set_scratch = [
        pltpu.VMEM((GB,), jnp.int32),
        pltpu.VMEM((GB,), jnp.int32),
        pltpu.VMEM((GB, C), jnp.float32),
        pltpu.VMEM((GB, C), jnp.float32),
        pltpu.SemaphoreType.DMA,
        pltpu.SemaphoreType.DMA,
        pltpu.SemaphoreType.DMA,
    ]
    fn = pl.kernel(
        _gather_k_body,
        out_type=[jax.ShapeDtypeStruct((E, C), jnp.float32)] * 2,
        mesh=mesh,
        scratch_types=set_scratch + set_scratch,
    )
    return fn(dst, src, q, ka, kb)


def _sc_gather_m(dst, src, va, vb):
    mesh = plsc.VectorSubcoreMesh(core_axis_name="c", subcore_axis_name="s",
                                  num_cores=NC, num_subcores=NS)
    set_scratch = [
        pltpu.VMEM((GB,), jnp.int32),
        pltpu.VMEM((GB,), jnp.int32),
        pltpu.VMEM((GB, C), jnp.float32),
        pltpu.SemaphoreType.DMA,
        pltpu.SemaphoreType.DMA,
    ]
    fn = pl.kernel(
        _gather_m_body,
        out_type=jax.ShapeDtypeStruct((E, C), jnp.float32),
        mesh=mesh,
        scratch_types=set_scratch + set_scratch,
    )
    return fn(dst, src, va, vb)


def _pass1_body(ghk, gq, eat, wek, ck, wku2, bku2, alpha_o, stats_o):
    i = pl.program_id(0)

    eak = lax.dot_general(eat[...], wek[...], (((0,), (0,)), ((), ())),
                          preferred_element_type=jnp.float32)
    h = ghk[...] + eak + ck[...]
    h = h * _sig(h)
    kj = jnp.dot(h, wku2[...], preferred_element_type=jnp.float32) + bku2[...]
    alpha = gq[...] * kj * INV_SQRT_C
    alpha_o[...] = alpha

    @pl.when(i == 0)
    def _():
        stats_o[...] = jnp.zeros_like(stats_o)

    s1 = jnp.sum(alpha, axis=0, keepdims=True)
    s2 = jnp.sum(alpha * alpha, axis=0, keepdims=True)
    stats_o[0:1, :] += s1
    stats_o[1:2, :] += s2


def _pass1(ghk, gq, eat, wek, ck, wku2, bku2):
    full = lambda shape: pl.BlockSpec(shape, lambda i: (0,) * len(shape))
    return pl.pallas_call(
        _pass1_body,
        grid=(E // BE,),
        in_specs=[
            pl.BlockSpec((BE, C), lambda i: (i, 0)),
            pl.BlockSpec((BE, C), lambda i: (i, 0)),
            pl.BlockSpec((ED, BE), lambda i: (0, i)),
            full((ED, C)), full((1, C)), full((C, C)), full((1, C)),
        ],
        out_specs=[pl.BlockSpec((BE, C), lambda i: (i, 0)),
                   pl.BlockSpec((8, C), lambda i: (0, 0))],
        out_shape=[jax.ShapeDtypeStruct((E, C), jnp.float32),
                   jax.ShapeDtypeStruct((8, C), jnp.float32)],
    )(ghk, gq, eat, wek, ck, wku2, bku2)


def _pass2_body(alpha, ghm, eat, stats, gatt, batt, wem, cm, wm2, bm2,
                gmsg_o):
    mu = stats[0:1, :] * (1.0 / E)
    ex2 = stats[1:2, :] * (1.0 / E)
    var = ex2 - mu * mu
    inv = gatt[...] * lax.rsqrt(var + BN_EPS)
    shift = batt[...] - mu * inv
    gate = _sig(alpha[...] * inv + shift)

    eam = lax.dot_general(eat[...], wem[...], (((0,), (0,)), ((), ())),
                          preferred_element_type=jnp.float32)
    h = ghm[...] + eam + cm[...]
    h = h * _sig(h)
    msg = jnp.dot(h, wm2[...], preferred_element_type=jnp.float32) + bm2[...]
    gmsg_o[...] = msg * gate


def _pass2(alpha, ghm, eat, stats, gatt, batt, wem, cm, wm2, bm2):
    full = lambda shape: pl.BlockSpec(shape, lambda i: (0,) * len(shape))
    return pl.pallas_call(
        _pass2_body,
        grid=(E // BE,),
        in_specs=[
            pl.BlockSpec((BE, C), lambda i: (i, 0)),
            pl.BlockSpec((BE, C), lambda i: (i, 0)),
            pl.BlockSpec((ED, BE), lambda i: (0, i)),
            full((8, C)), full((1, C)), full((1, C)),
            full((ED, C)), full((1, C)), full((C, C)), full((1, C)),
        ],
        out_specs=pl.BlockSpec((BE, C), lambda i: (i, 0)),
        out_shape=jax.ShapeDtypeStruct((E, C), jnp.float32),
    )(alpha, ghm, eat, stats, gatt, batt, wem, cm, wm2, bm2)


def _sc_scatter_body(dst_h, gmsg_h, zeros_h, parts_h, idxb, datab, acc):
    cid = lax.axis_index("c")
    sid = lax.axis_index("s")
    wid = sid * NC + cid
    base = wid * EPW
    r0 = sid * INIT_ROWS

    @pl.when(sid < N // INIT_ROWS)
    def _():
        pltpu.sync_copy(zeros_h.at[pl.ds(r0, INIT_ROWS)],
                        acc.at[pl.ds(r0, INIT_ROWS)])
    plsc.subcore_barrier()

    def chunk(c, carry):
        off = base + c * SB
        pltpu.sync_copy(dst_h.at[pl.ds(off, SB)], idxb)
        pltpu.sync_copy(gmsg_h.at[pl.ds(off, SB)], datab)
        pltpu.sync_copy(datab, acc.at[idxb], add=True)
        return carry

    lax.fori_loop(0, EPW // SB, chunk, 0)
    plsc.subcore_barrier()

    @pl.when(sid < N // INIT_ROWS)
    def _():
        pltpu.sync_copy(acc.at[pl.ds(r0, INIT_ROWS)],
                        parts_h.at[cid, pl.ds(r0, INIT_ROWS)])


def _sc_scatter(dst, gmsg, zeros):
    mesh = plsc.VectorSubcoreMesh(core_axis_name="c", subcore_axis_name="s",
                                  num_cores=NC, num_subcores=NS)
    fn = pl.kernel(
        _sc_scatter_body,
        out_type=jax.ShapeDtypeStruct((NC, N, C), jnp.float32),
        mesh=mesh,
        scratch_types=[
            pltpu.VMEM((SB,), jnp.int32),
            pltpu.VMEM((SB, C), jnp.float32),
            pltpu.VMEM_SHARED((N, C), jnp.float32),
        ],
    )
    return fn(dst, gmsg, zeros)


def _final_body(parts, x_ref, wc, bc, gbn, bbn, out_o):
    agg = parts[0] + parts[1]
    o = jnp.dot(agg, wc[...], preferred_element_type=jnp.float32) + bc[...]
    mu = jnp.sum(o, axis=0, keepdims=True) * (1.0 / N)
    ex2 = jnp.sum(o * o, axis=0, keepdims=True) * (1.0 / N)
    var = ex2 - mu * mu
    inv = gbn[...] * lax.rsqrt(var + BN_EPS)
    z = x_ref[...] + o * inv + (bbn[...] - mu * inv)
    out_o[...] = jnp.maximum(z, 0.0) + jnp.log(1.0 + jnp.exp(-jnp.abs(z)))


def _final(parts, x, wc, bc, gbn, bbn):
    return pl.pallas_call(
        _final_body,
        out_shape=jax.ShapeDtypeStruct((N, C), jnp.float32),
    )(parts, x, wc, bc.reshape(1, C), gbn.reshape(1, C), bbn.reshape(1, C))


def kernel(x, edge_index, edge_attr, params):
    p = params
    ei = edge_index.astype(jnp.int32)
    src = ei[0]
    dst = ei[1]

    wek = p['We'] @ p['Wku1'][2 * C:]
    ck = (p['bku1'] + p['be'] @ p['Wku1'][2 * C:]).reshape(1, C)
    wem = p['We'] @ p['Wm1'][2 * C:]
    cm = (p['bm1'] + p['be'] @ p['Wm1'][2 * C:]).reshape(1, C)

    eat = edge_attr.T
    q, ka, kb, va, vb = _node_tables(x, p)
    gq, ghk = _sc_gather_k(dst, src, q, ka, kb)
    ghm = _sc_gather_m(dst, src, va, vb)
    alpha, stats = _pass1(ghk, gq, eat, wek, ck, p['Wku2'],
                          p['bku2'].reshape(1, C))
    gmsg = _pass2(alpha, ghm, eat, stats,
                  p['g_att'].reshape(1, C), p['b_att'].reshape(1, C),
                  wem, cm, p['Wm2'], p['bm2'].reshape(1, C))
    zeros = jnp.zeros((N, C), jnp.float32)
    parts = _sc_scatter(dst, gmsg, zeros)
    return _final(parts, x, p['Wc'], p['bc'], p['g_bn'], p['b_bn'])


# speedup vs baseline: 4.6116x; 1.0756x over previous
import functools

import jax
import jax.numpy as jnp
import numpy as np
from jax import lax
from jax.experimental import pallas as pl
from jax.experimental.pallas import tpu as pltpu
from jax.experimental.pallas import tpu_sc as plsc

N = 10000
E = 320000
D = 128
ED = 16
C = 128

NC = 2
NS = 16
NW = NC * NS
EPW = E // NW
GB = 200
SB = 200
INIT_ROWS = 1000

BE = 3200
INV_SQRT_C = 1.0 / float(np.sqrt(C))
BN_EPS = 1e-5


def _sig(z):
    return 1.0 / (1.0 + jnp.exp(-z))


def _node_tables_body(x_ref, wq, bq, wk, bk, wv, bv, wku1, wm1,
                      q_o, ka_o, kb_o, va_o, vb_o):
    xb = x_ref[...]
    q = jnp.dot(xb, wq[...], preferred_element_type=jnp.float32) + bq[...]
    k = jnp.dot(xb, wk[...], preferred_element_type=jnp.float32) + bk[...]
    v = jnp.dot(xb, wv[...], preferred_element_type=jnp.float32) + bv[...]
    q_o[...] = q
    ka_o[...] = jnp.dot(k, wku1[0:C, :], preferred_element_type=jnp.float32)
    kb_o[...] = jnp.dot(k, wku1[C:2 * C, :], preferred_element_type=jnp.float32)
    va_o[...] = jnp.dot(v, wm1[0:C, :], preferred_element_type=jnp.float32)
    vb_o[...] = jnp.dot(v, wm1[C:2 * C, :], preferred_element_type=jnp.float32)


def _node_tables(x, p):
    Bn = 2000
    full = lambda shape: pl.BlockSpec(shape, lambda i: (0,) * len(shape))
    return pl.pallas_call(
        _node_tables_body,
        grid=(N // Bn,),
        in_specs=[
            pl.BlockSpec((Bn, D), lambda i: (i, 0)),
            full((D, C)), full((1, C)),
            full((D, C)), full((1, C)),
            full((D, C)), full((1, C)),
            full((3 * C, C)), full((3 * C, C)),
        ],
        out_specs=[pl.BlockSpec((Bn, C), lambda i: (i, 0))] * 5,
        out_shape=[jax.ShapeDtypeStruct((N, C), jnp.float32)] * 5,
    )(x, p['Wq'], p['bq'].reshape(1, C), p['Wk'], p['bk'].reshape(1, C),
      p['Wv'], p['bv'].reshape(1, C), p['Wku1'], p['Wm1'])


def _gather_k_body(dst_h, src_h, q_h, ka_h, kb_h, gq_h, ghk_h,
                   di0, si0, bq0, bhk0, qsem0, ksem0, osem0,
                   di1, si1, bq1, bhk1, qsem1, ksem1, osem1):
    wid = lax.axis_index("s") * NC + lax.axis_index("c")
    base = wid * EPW
    nch = EPW // GB
    sets = ((di0, si0, bq0, bhk0, qsem0, ksem0, osem0),
            (di1, si1, bq1, bhk1, qsem1, ksem1, osem1))

    def stage1(j, st):
        di, si, bq, bhk, qsem, ksem, osem = st
        off = base + j * GB

        @pl.when(j >= 2)
        def _():
            pltpu.make_async_copy(bq, gq_h.at[pl.ds(off, GB)], osem).wait()
            pltpu.make_async_copy(bhk, ghk_h.at[pl.ds(off, GB)], osem).wait()

        pltpu.sync_copy(dst_h.at[pl.ds(off, GB)], di)
        pltpu.sync_copy(src_h.at[pl.ds(off, GB)], si)
        pltpu.async_copy(q_h.at[di], bq, qsem)
        pltpu.async_copy(ka_h.at[di], bhk, ksem)

    def finish(c, s):
        di, si, bq, bhk, qsem, ksem, osem = sets[s]
        off = base + c * GB
        pltpu.make_async_copy(ka_h.at[pl.ds(0, GB)], bhk, ksem).wait()
        kb_cp = pltpu.async_copy(kb_h.at[si], bhk, ksem, add=True)

        @pl.when(c + 1 < nch)
        def _():
            stage1(c + 1, sets[1 - s])

        pltpu.make_async_copy(q_h.at[pl.ds(0, GB)], bq, qsem).wait()
        pltpu.async_copy(bq, gq_h.at[pl.ds(off, GB)], osem)
        kb_cp.wait()
        pltpu.async_copy(bhk, ghk_h.at[pl.ds(off, GB)], osem)

    stage1(0, sets[0])

    def pair(p, carry):
        finish(2 * p, 0)
        finish(2 * p + 1, 1)
        return carry

    lax.fori_loop(0, nch // 2, pair, 0)
    for s in (0, 1):
        di, si, bq, bhk, qsem, ksem, osem = sets[s]
        pltpu.make_async_copy(bq, gq_h.at[pl.ds(base, GB)], osem).wait()
        pltpu.make_async_copy(bhk, ghk_h.at[pl.ds(base, GB)], osem).wait()


def _gather_m_body(dst_h, src_h, va_h, vb_h, ghm_h,
                   di0, si0, bhm0, gsem0, osem0,
                   di1, si1, bhm1, gsem1, osem1):
    wid = lax.axis_index("s") * NC + lax.axis_index("c")
    base = wid * EPW
    nch = EPW // GB
    sets = ((di0, si0, bhm0, gsem0, osem0),
            (di1, si1, bhm1, gsem1, osem1))

    def stage1(j, st):
        di, si, bhm, gsem, osem = st
        off = base + j * GB

        @pl.when(j >= 2)
        def _():
            pltpu.make_async_copy(bhm, ghm_h.at[pl.ds(off, GB)], osem).wait()

        pltpu.sync_copy(dst_h.at[pl.ds(off, GB)], di)
        pltpu.sync_copy(src_h.at[pl.ds(off, GB)], si)
        pltpu.async_copy(va_h.at[di], bhm, gsem)

    def finish(c, s):
        di, si, bhm, gsem, osem = sets[s]
        off = base + c * GB
        pltpu.make_async_copy(va_h.at[pl.ds(0, GB)], bhm, gsem).wait()
        vb_cp = pltpu.async_copy(vb_h.at[si], bhm, gsem, add=True)

        @pl.when(c + 1 < nch)
        def _():
            stage1(c + 1, sets[1 - s])

        vb_cp.wait()
        pltpu.async_copy(bhm, ghm_h.at[pl.ds(off, GB)], osem)

    stage1(0, sets[0])

    def pair(p, carry):
        finish(2 * p, 0)
        finish(2 * p + 1, 1)
        return carry

    lax.fori_loop(0, nch // 2, pair, 0)
    for s in (0, 1):
        di, si, bhm, gsem, osem = sets[s]
        pltpu.make_async_copy(bhm, ghm_h.at[pl.ds(base, GB)], osem).wait()


def _sc_gather_k(dst, src, q, ka, kb):
    mesh = plsc.VectorSubcoreMesh(core_axis_name="c", subcore_axis_name="s",
                                  num_cores=NC, num_subcores=NS)
    set_scratch = [
        pltpu.VMEM((GB,), jnp.int32),
        pltpu.VMEM((GB,), jnp.int32),
        pltpu.VMEM((GB, C), jnp.float32),
        pltpu.VMEM((GB, C), jnp.float32),
        pltpu.SemaphoreType.DMA,
        pltpu.SemaphoreType.DMA,
        pltpu.SemaphoreType.DMA,
    ]
    fn = pl.kernel(
        _gather_k_body,
        out_type=[jax.ShapeDtypeStruct((E, C), jnp.float32)] * 2,
        mesh=mesh,
        scratch_types=set_scratch + set_scratch,
    )
    return fn(dst, src, q, ka, kb)


def _sc_gather_m(dst, src, va, vb):
    mesh = plsc.VectorSubcoreMesh(core_axis_name="c", subcore_axis_name="s",
                                  num_cores=NC, num_subcores=NS)
    set_scratch = [
        pltpu.VMEM((GB,), jnp.int32),
        pltpu.VMEM((GB,), jnp.int32),
        pltpu.VMEM((GB, C), jnp.float32),
        pltpu.SemaphoreType.DMA,
        pltpu.SemaphoreType.DMA,
    ]
    fn = pl.kernel(
        _gather_m_body,
        out_type=jax.ShapeDtypeStruct((E, C), jnp.float32),
        mesh=mesh,
        scratch_types=set_scratch + set_scratch,
    )
    return fn(dst, src, va, vb)


def _pass1_body(ghk, gq, eat, wek, ck, wku2, bku2, alpha_o, stats_o):
    i = pl.program_id(0)

    eak = lax.dot_general(eat[...], wek[...], (((0,), (0,)), ((), ())),
                          preferred_element_type=jnp.float32)
    h = ghk[...] + eak + ck[...]
    h = h * _sig(h)
    kj = jnp.dot(h, wku2[...], preferred_element_type=jnp.float32) + bku2[...]
    alpha = gq[...] * kj * INV_SQRT_C
    alpha_o[...] = alpha

    @pl.when(i == 0)
    def _():
        stats_o[...] = jnp.zeros_like(stats_o)

    s1 = jnp.sum(alpha, axis=0, keepdims=True)
    s2 = jnp.sum(alpha * alpha, axis=0, keepdims=True)
    stats_o[0:1, :] += s1
    stats_o[1:2, :] += s2


def _pass1(ghk, gq, eat, wek, ck, wku2, bku2):
    full = lambda shape: pl.BlockSpec(shape, lambda i: (0,) * len(shape))
    return pl.pallas_call(
        _pass1_body,
        grid=(E // BE,),
        in_specs=[
            pl.BlockSpec((BE, C), lambda i: (i, 0)),
            pl.BlockSpec((BE, C), lambda i: (i, 0)),
            pl.BlockSpec((ED, BE), lambda i: (0, i)),
            full((ED, C)), full((1, C)), full((C, C)), full((1, C)),
        ],
        out_specs=[pl.BlockSpec((BE, C), lambda i: (i, 0)),
                   pl.BlockSpec((8, C), lambda i: (0, 0))],
        out_shape=[jax.ShapeDtypeStruct((E, C), jnp.float32),
                   jax.ShapeDtypeStruct((8, C), jnp.float32)],
    )(ghk, gq, eat, wek, ck, wku2, bku2)


def _pass2_body(alpha, ghm, eat, stats, gatt, batt, wem, cm, wm2, bm2,
                gmsg_o):
    mu = stats[0:1, :] * (1.0 / E)
    ex2 = stats[1:2, :] * (1.0 / E)
    var = ex2 - mu * mu
    inv = gatt[...] * lax.rsqrt(var + BN_EPS)
    shift = batt[...] - mu * inv
    gate = _sig(alpha[...] * inv + shift)

    eam = lax.dot_general(eat[...], wem[...], (((0,), (0,)), ((), ())),
                          preferred_element_type=jnp.float32)
    h = ghm[...] + eam + cm[...]
    h = h * _sig(h)
    msg = jnp.dot(h, wm2[...], preferred_element_type=jnp.float32) + bm2[...]
    gmsg_o[...] = msg * gate


def _pass2(alpha, ghm, eat, stats, gatt, batt, wem, cm, wm2, bm2):
    full = lambda shape: pl.BlockSpec(shape, lambda i: (0,) * len(shape))
    return pl.pallas_call(
        _pass2_body,
        grid=(E // BE,),
        in_specs=[
            pl.BlockSpec((BE, C), lambda i: (i, 0)),
            pl.BlockSpec((BE, C), lambda i: (i, 0)),
            pl.BlockSpec((ED, BE), lambda i: (0, i)),
            full((8, C)), full((1, C)), full((1, C)),
            full((ED, C)), full((1, C)), full((C, C)), full((1, C)),
        ],
        out_specs=pl.BlockSpec((BE, C), lambda i: (i, 0)),
        out_shape=jax.ShapeDtypeStruct((E, C), jnp.float32),
    )(alpha, ghm, eat, stats, gatt, batt, wem, cm, wm2, bm2)


def _sc_scatter_body(dst_h, gmsg_h, zeros_h, parts_h, idxb, datab, acc):
    cid = lax.axis_index("c")
    sid = lax.axis_index("s")
    wid = sid * NC + cid
    base = wid * EPW
    r0 = sid * INIT_ROWS

    @pl.when(sid < N // INIT_ROWS)
    def _():
        pltpu.sync_copy(zeros_h.at[pl.ds(r0, INIT_ROWS)],
                        acc.at[pl.ds(r0, INIT_ROWS)])
    plsc.subcore_barrier()

    def chunk(c, carry):
        off = base + c * SB
        pltpu.sync_copy(dst_h.at[pl.ds(off, SB)], idxb)
        pltpu.sync_copy(gmsg_h.at[pl.ds(off, SB)], datab)
        pltpu.sync_copy(datab, acc.at[idxb], add=True)
        return carry

    lax.fori_loop(0, EPW // SB, chunk, 0)
    plsc.subcore_barrier()

    @pl.when(sid < N // INIT_ROWS)
    def _():
        pltpu.sync_copy(acc.at[pl.ds(r0, INIT_ROWS)],
                        parts_h.at[cid, pl.ds(r0, INIT_ROWS)])


def _sc_scatter(dst, gmsg, zeros):
    mesh = plsc.VectorSubcoreMesh(core_axis_name="c", subcore_axis_name="s",
                                  num_cores=NC, num_subcores=NS)
    fn = pl.kernel(
        _sc_scatter_body,
        out_type=jax.ShapeDtypeStruct((NC, N, C), jnp.float32),
        mesh=mesh,
        scratch_types=[
            pltpu.VMEM((SB,), jnp.int32),
            pltpu.VMEM((SB, C), jnp.float32),
            pltpu.VMEM_SHARED((N, C), jnp.float32),
        ],
    )
    return fn(dst, gmsg, zeros)


def _final_body(parts, x_ref, wc, bc, gbn, bbn, out_o):
    agg = parts[0] + parts[1]
    o = jnp.dot(agg, wc[...], preferred_element_type=jnp.float32) + bc[...]
    mu = jnp.sum(o, axis=0, keepdims=True) * (1.0 / N)
    ex2 = jnp.sum(o * o, axis=0, keepdims=True) * (1.0 / N)
    var = ex2 - mu * mu
    inv = gbn[...] * lax.rsqrt(var + BN_EPS)
    z = x_ref[...] + o * inv + (bbn[...] - mu * inv)
    out_o[...] = jnp.maximum(z, 0.0) + jnp.log(1.0 + jnp.exp(-jnp.abs(z)))


def _final(parts, x, wc, bc, gbn, bbn):
    return pl.pallas_call(
        _final_body,
        out_shape=jax.ShapeDtypeStruct((N, C), jnp.float32),
    )(parts, x, wc, bc.reshape(1, C), gbn.reshape(1, C), bbn.reshape(1, C))


def kernel(x, edge_index, edge_attr, params):
    p = params
    ei = edge_index.astype(jnp.int32)
    src = ei[0]
    dst = ei[1]

    wek = p['We'] @ p['Wku1'][2 * C:]
    ck = (p['bku1'] + p['be'] @ p['Wku1'][2 * C:]).reshape(1, C)
    wem = p['We'] @ p['Wm1'][2 * C:]
    cm = (p['bm1'] + p['be'] @ p['Wm1'][2 * C:]).reshape(1, C)

    eat = edge_attr.T
    q, ka, kb, va, vb = _node_tables(x, p)
    gq, ghk = _sc_gather_k(dst, src, q, ka, kb)
    ghm = _sc_gather_m(dst, src, va, vb)
    alpha, stats = _pass1(ghk, gq, eat, wek, ck, p['Wku2'],
                          p['bku2'].reshape(1, C))
    gmsg = _pass2(alpha, ghm, eat, stats,
                  p['g_att'].reshape(1, C), p['b_att'].reshape(1, C),
                  wem, cm, p['Wm2'], p['bm2'].reshape(1, C))
    zeros = jnp.zeros((N, C), jnp.float32)
    parts = _sc_scatter(dst, gmsg, zeros)
    return _final(parts, x, p['Wc'], p['bc'], p['g_bn'], p['b_bn'])


# speedup vs baseline: 4.7091x; 1.0211x over previous
import functools

import jax
import jax.numpy as jnp
import numpy as np
from jax import lax
from jax.experimental import pallas as pl
from jax.experimental.pallas import tpu as pltpu
from jax.experimental.pallas import tpu_sc as plsc

N = 10000
E = 320000
D = 128
ED = 16
C = 128

NC = 2
NS = 16
NW = NC * NS
EPW = E // NW
GB = 200
SB = 80
INIT_ROWS = 1000

BE = 3200
INV_SQRT_C = 1.0 / float(np.sqrt(C))
BN_EPS = 1e-5


def _sig(z):
    return 1.0 / (1.0 + jnp.exp(-z))


def _node_tables_body(x_ref, wq, bq, wk, bk, wv, bv, wku1, wm1,
                      q_o, ka_o, kb_o, va_o, vb_o):
    xb = x_ref[...]
    q = jnp.dot(xb, wq[...], preferred_element_type=jnp.float32) + bq[...]
    k = jnp.dot(xb, wk[...], preferred_element_type=jnp.float32) + bk[...]
    v = jnp.dot(xb, wv[...], preferred_element_type=jnp.float32) + bv[...]
    q_o[...] = q
    ka_o[...] = jnp.dot(k, wku1[0:C, :], preferred_element_type=jnp.float32)
    kb_o[...] = jnp.dot(k, wku1[C:2 * C, :], preferred_element_type=jnp.float32)
    va_o[...] = jnp.dot(v, wm1[0:C, :], preferred_element_type=jnp.float32)
    vb_o[...] = jnp.dot(v, wm1[C:2 * C, :], preferred_element_type=jnp.float32)


def _node_tables(x, p):
    Bn = 2000
    full = lambda shape: pl.BlockSpec(shape, lambda i: (0,) * len(shape))
    return pl.pallas_call(
        _node_tables_body,
        grid=(N // Bn,),
        in_specs=[
            pl.BlockSpec((Bn, D), lambda i: (i, 0)),
            full((D, C)), full((1, C)),
            full((D, C)), full((1, C)),
            full((D, C)), full((1, C)),
            full((3 * C, C)), full((3 * C, C)),
        ],
        out_specs=[pl.BlockSpec((Bn, C), lambda i: (i, 0))] * 5,
        out_shape=[jax.ShapeDtypeStruct((N, C), jnp.float32)] * 5,
    )(x, p['Wq'], p['bq'].reshape(1, C), p['Wk'], p['bk'].reshape(1, C),
      p['Wv'], p['bv'].reshape(1, C), p['Wku1'], p['Wm1'])


def _gather_k_body(dst_h, src_h, q_h, ka_h, kb_h, gq_h, ghk_h,
                   di0, si0, bq0, bhk0, qsem0, ksem0, osem0,
                   di1, si1, bq1, bhk1, qsem1, ksem1, osem1):
    wid = lax.axis_index("s") * NC + lax.axis_index("c")
    base = wid * EPW
    nch = EPW // GB
    sets = ((di0, si0, bq0, bhk0, qsem0, ksem0, osem0),
            (di1, si1, bq1, bhk1, qsem1, ksem1, osem1))

    def stage1(j, st):
        di, si, bq, bhk, qsem, ksem, osem = st
        off = base + j * GB

        @pl.when(j >= 2)
        def _():
            pltpu.make_async_copy(bq, gq_h.at[pl.ds(off, GB)], osem).wait()
            pltpu.make_async_copy(bhk, ghk_h.at[pl.ds(off, GB)], osem).wait()

        pltpu.sync_copy(dst_h.at[pl.ds(off, GB)], di)
        pltpu.sync_copy(src_h.at[pl.ds(off, GB)], si)
        pltpu.async_copy(q_h.at[di], bq, qsem)
        pltpu.async_copy(ka_h.at[di], bhk, ksem)

    def finish(c, s):
        di, si, bq, bhk, qsem, ksem, osem = sets[s]
        off = base + c * GB
        pltpu.make_async_copy(ka_h.at[pl.ds(0, GB)], bhk, ksem).wait()
        kb_cp = pltpu.async_copy(kb_h.at[si], bhk, ksem, add=True)

        @pl.when(c + 1 < nch)
        def _():
            stage1(c + 1, sets[1 - s])

        pltpu.make_async_copy(q_h.at[pl.ds(0, GB)], bq, qsem).wait()
        pltpu.async_copy(bq, gq_h.at[pl.ds(off, GB)], osem)
        kb_cp.wait()
        pltpu.async_copy(bhk, ghk_h.at[pl.ds(off, GB)], osem)

    stage1(0, sets[0])

    def pair(p, carry):
        finish(2 * p, 0)
        finish(2 * p + 1, 1)
        return carry

    lax.fori_loop(0, nch // 2, pair, 0)
    for s in (0, 1):
        di, si, bq, bhk, qsem, ksem, osem = sets[s]
        pltpu.make_async_copy(bq, gq_h.at[pl.ds(base, GB)], osem).wait()
        pltpu.make_async_copy(bhk, ghk_h.at[pl.ds(base, GB)], osem).wait()


def _gather_m_body(dst_h, src_h, va_h, vb_h, ghm_h,
                   di0, si0, bhm0, gsem0, osem0,
                   di1, si1, bhm1, gsem1, osem1):
    wid = lax.axis_index("s") * NC + lax.axis_index("c")
    base = wid * EPW
    nch = EPW // GB
    sets = ((di0, si0, bhm0, gsem0, osem0),
            (di1, si1, bhm1, gsem1, osem1))

    def stage1(j, st):
        di, si, bhm, gsem, osem = st
        off = base + j * GB

        @pl.when(j >= 2)
        def _():
            pltpu.make_async_copy(bhm, ghm_h.at[pl.ds(off, GB)], osem).wait()

        pltpu.sync_copy(dst_h.at[pl.ds(off, GB)], di)
        pltpu.sync_copy(src_h.at[pl.ds(off, GB)], si)
        pltpu.async_copy(va_h.at[di], bhm, gsem)

    def finish(c, s):
        di, si, bhm, gsem, osem = sets[s]
        off = base + c * GB
        pltpu.make_async_copy(va_h.at[pl.ds(0, GB)], bhm, gsem).wait()
        vb_cp = pltpu.async_copy(vb_h.at[si], bhm, gsem, add=True)

        @pl.when(c + 1 < nch)
        def _():
            stage1(c + 1, sets[1 - s])

        vb_cp.wait()
        pltpu.async_copy(bhm, ghm_h.at[pl.ds(off, GB)], osem)

    stage1(0, sets[0])

    def pair(p, carry):
        finish(2 * p, 0)
        finish(2 * p + 1, 1)
        return carry

    lax.fori_loop(0, nch // 2, pair, 0)
    for s in (0, 1):
        di, si, bhm, gsem, osem = sets[s]
        pltpu.make_async_copy(bhm, ghm_h.at[pl.ds(base, GB)], osem).wait()


def _sc_gather_k(dst, src, q, ka, kb):
    mesh = plsc.VectorSubcoreMesh(core_axis_name="c", subcore_axis_name="s",
                                  num_cores=NC, num_subcores=NS)
    set_scratch = [
        pltpu.VMEM((GB,), jnp.int32),
        pltpu.VMEM((GB,), jnp.int32),
        pltpu.VMEM((GB, C), jnp.float32),
        pltpu.VMEM((GB, C), jnp.float32),
        pltpu.SemaphoreType.DMA,
        pltpu.SemaphoreType.DMA,
        pltpu.SemaphoreType.DMA,
    ]
    fn = pl.kernel(
        _gather_k_body,
        out_type=[jax.ShapeDtypeStruct((E, C), jnp.float32)] * 2,
        mesh=mesh,
        scratch_types=set_scratch + set_scratch,
    )
    return fn(dst, src, q, ka, kb)


def _sc_gather_m(dst, src, va, vb):
    mesh = plsc.VectorSubcoreMesh(core_axis_name="c", subcore_axis_name="s",
                                  num_cores=NC, num_subcores=NS)
    set_scratch = [
        pltpu.VMEM((GB,), jnp.int32),
        pltpu.VMEM((GB,), jnp.int32),
        pltpu.VMEM((GB, C), jnp.float32),
        pltpu.SemaphoreType.DMA,
        pltpu.SemaphoreType.DMA,
    ]
    fn = pl.kernel(
        _gather_m_body,
        out_type=jax.ShapeDtypeStruct((E, C), jnp.float32),
        mesh=mesh,
        scratch_types=set_scratch + set_scratch,
    )
    return fn(dst, src, va, vb)


def _pass1_body(ghk, gq, eat, wek, ck, wku2, bku2, alpha_o, stats_o):
    i = pl.program_id(0)

    eak = lax.dot_general(eat[...], wek[...], (((0,), (0,)), ((), ())),
                          preferred_element_type=jnp.float32)
    h = ghk[...] + eak + ck[...]
    h = h * _sig(h)
    kj = jnp.dot(h, wku2[...], preferred_element_type=jnp.float32) + bku2[...]
    alpha = gq[...] * kj * INV_SQRT_C
    alpha_o[...] = alpha

    @pl.when(i == 0)
    def _():
        stats_o[...] = jnp.zeros_like(stats_o)

    s1 = jnp.sum(alpha, axis=0, keepdims=True)
    s2 = jnp.sum(alpha * alpha, axis=0, keepdims=True)
    stats_o[0:1, :] += s1
    stats_o[1:2, :] += s2


def _pass1(ghk, gq, eat, wek, ck, wku2, bku2):
    full = lambda shape: pl.BlockSpec(shape, lambda i: (0,) * len(shape))
    return pl.pallas_call(
        _pass1_body,
        grid=(E // BE,),
        in_specs=[
            pl.BlockSpec((BE, C), lambda i: (i, 0)),
            pl.BlockSpec((BE, C), lambda i: (i, 0)),
            pl.BlockSpec((ED, BE), lambda i: (0, i)),
            full((ED, C)), full((1, C)), full((C, C)), full((1, C)),
        ],
        out_specs=[pl.BlockSpec((BE, C), lambda i: (i, 0)),
                   pl.BlockSpec((8, C), lambda i: (0, 0))],
        out_shape=[jax.ShapeDtypeStruct((E, C), jnp.float32),
                   jax.ShapeDtypeStruct((8, C), jnp.float32)],
    )(ghk, gq, eat, wek, ck, wku2, bku2)


def _pass2_body(alpha, ghm, eat, stats, gatt, batt, wem, cm, wm2, bm2,
                gmsg_o):
    mu = stats[0:1, :] * (1.0 / E)
    ex2 = stats[1:2, :] * (1.0 / E)
    var = ex2 - mu * mu
    inv = gatt[...] * lax.rsqrt(var + BN_EPS)
    shift = batt[...] - mu * inv
    gate = _sig(alpha[...] * inv + shift)

    eam = lax.dot_general(eat[...], wem[...], (((0,), (0,)), ((), ())),
                          preferred_element_type=jnp.float32)
    h = ghm[...] + eam + cm[...]
    h = h * _sig(h)
    msg = jnp.dot(h, wm2[...], preferred_element_type=jnp.float32) + bm2[...]
    gmsg_o[...] = msg * gate


def _pass2(alpha, ghm, eat, stats, gatt, batt, wem, cm, wm2, bm2):
    full = lambda shape: pl.BlockSpec(shape, lambda i: (0,) * len(shape))
    return pl.pallas_call(
        _pass2_body,
        grid=(E // BE,),
        in_specs=[
            pl.BlockSpec((BE, C), lambda i: (i, 0)),
            pl.BlockSpec((BE, C), lambda i: (i, 0)),
            pl.BlockSpec((ED, BE), lambda i: (0, i)),
            full((8, C)), full((1, C)), full((1, C)),
            full((ED, C)), full((1, C)), full((C, C)), full((1, C)),
        ],
        out_specs=pl.BlockSpec((BE, C), lambda i: (i, 0)),
        out_shape=jax.ShapeDtypeStruct((E, C), jnp.float32),
    )(alpha, ghm, eat, stats, gatt, batt, wem, cm, wm2, bm2)


def _sc_scatter_body(dst_h, gmsg_h, zeros_h, parts_h,
                     idx0, dat0, lsem0, ssem0,
                     idx1, dat1, lsem1, ssem1, acc):
    cid = lax.axis_index("c")
    sid = lax.axis_index("s")
    wid = sid * NC + cid
    base = wid * EPW
    nch = EPW // SB
    r0 = sid * INIT_ROWS
    sets = ((idx0, dat0, lsem0, ssem0), (idx1, dat1, lsem1, ssem1))

    @pl.when(sid < N // INIT_ROWS)
    def _():
        pltpu.sync_copy(zeros_h.at[pl.ds(r0, INIT_ROWS)],
                        acc.at[pl.ds(r0, INIT_ROWS)])
    plsc.subcore_barrier()

    def stage1(j, st):
        idxb, datab, lsem, ssem = st

        @pl.when(j >= 2)
        def _():
            pltpu.make_async_copy(datab, acc.at[pl.ds(0, SB)], ssem).wait()

        off = base + j * SB
        pltpu.async_copy(dst_h.at[pl.ds(off, SB)], idxb, lsem)
        pltpu.async_copy(gmsg_h.at[pl.ds(off, SB)], datab, lsem)

    def finish(c, s):
        idxb, datab, lsem, ssem = sets[s]
        pltpu.make_async_copy(dst_h.at[pl.ds(0, SB)], idxb, lsem).wait()
        pltpu.make_async_copy(gmsg_h.at[pl.ds(0, SB)], datab, lsem).wait()
        pltpu.async_copy(datab, acc.at[idxb], ssem, add=True)

        @pl.when(c + 1 < nch)
        def _():
            stage1(c + 1, sets[1 - s])

    stage1(0, sets[0])

    def pair(p, carry):
        finish(2 * p, 0)
        finish(2 * p + 1, 1)
        return carry

    lax.fori_loop(0, nch // 2, pair, 0)
    if nch % 2:
        finish(nch - 1, (nch - 1) % 2)
    for s in (0, 1):
        idxb, datab, lsem, ssem = sets[s]
        pltpu.make_async_copy(datab, acc.at[pl.ds(0, SB)], ssem).wait()
    plsc.subcore_barrier()

    @pl.when(sid < N // INIT_ROWS)
    def _():
        pltpu.sync_copy(acc.at[pl.ds(r0, INIT_ROWS)],
                        parts_h.at[cid, pl.ds(r0, INIT_ROWS)])


def _sc_scatter(dst, gmsg, zeros):
    mesh = plsc.VectorSubcoreMesh(core_axis_name="c", subcore_axis_name="s",
                                  num_cores=NC, num_subcores=NS)
    set_scratch = [
        pltpu.VMEM((SB,), jnp.int32),
        pltpu.VMEM((SB, C), jnp.float32),
        pltpu.SemaphoreType.DMA,
        pltpu.SemaphoreType.DMA,
    ]
    fn = pl.kernel(
        _sc_scatter_body,
        out_type=jax.ShapeDtypeStruct((NC, N, C), jnp.float32),
        mesh=mesh,
        scratch_types=set_scratch + set_scratch + [
            pltpu.VMEM_SHARED((N, C), jnp.float32),
        ],
    )
    return fn(dst, gmsg, zeros)


def _final_body(parts, x_ref, wc, bc, gbn, bbn, out_o):
    agg = parts[0] + parts[1]
    o = jnp.dot(agg, wc[...], preferred_element_type=jnp.float32) + bc[...]
    mu = jnp.sum(o, axis=0, keepdims=True) * (1.0 / N)
    ex2 = jnp.sum(o * o, axis=0, keepdims=True) * (1.0 / N)
    var = ex2 - mu * mu
    inv = gbn[...] * lax.rsqrt(var + BN_EPS)
    z = x_ref[...] + o * inv + (bbn[...] - mu * inv)
    out_o[...] = jnp.maximum(z, 0.0) + jnp.log(1.0 + jnp.exp(-jnp.abs(z)))


def _final(parts, x, wc, bc, gbn, bbn):
    return pl.pallas_call(
        _final_body,
        out_shape=jax.ShapeDtypeStruct((N, C), jnp.float32),
    )(parts, x, wc, bc.reshape(1, C), gbn.reshape(1, C), bbn.reshape(1, C))


def kernel(x, edge_index, edge_attr, params):
    p = params
    ei = edge_index.astype(jnp.int32)
    src = ei[0]
    dst = ei[1]

    wek = p['We'] @ p['Wku1'][2 * C:]
    ck = (p['bku1'] + p['be'] @ p['Wku1'][2 * C:]).reshape(1, C)
    wem = p['We'] @ p['Wm1'][2 * C:]
    cm = (p['bm1'] + p['be'] @ p['Wm1'][2 * C:]).reshape(1, C)

    eat = edge_attr.T
    q, ka, kb, va, vb = _node_tables(x, p)
    gq, ghk = _sc_gather_k(dst, src, q, ka, kb)
    ghm = _sc_gather_m(dst, src, va, vb)
    alpha, stats = _pass1(ghk, gq, eat, wek, ck, p['Wku2'],
                          p['bku2'].reshape(1, C))
    gmsg = _pass2(alpha, ghm, eat, stats,
                  p['g_att'].reshape(1, C), p['b_att'].reshape(1, C),
                  wem, cm, p['Wm2'], p['bm2'].reshape(1, C))
    zeros = jnp.zeros((N, C), jnp.float32)
    parts = _sc_scatter(dst, gmsg, zeros)
    return _final(parts, x, p['Wc'], p['bc'], p['g_bn'], p['b_bn'])


# speedup vs baseline: 4.8799x; 1.0363x over previous
import functools

import jax
import jax.numpy as jnp
import numpy as np
from jax import lax
from jax.experimental import pallas as pl
from jax.experimental.pallas import tpu as pltpu
from jax.experimental.pallas import tpu_sc as plsc

N = 10000
E = 320000
D = 128
ED = 16
C = 128

NC = 2
NS = 16
NW = NC * NS
EPW = E // NW
GB = 200
SB = 80
INIT_ROWS = 1000

BE = 3200
INV_SQRT_C = 1.0 / float(np.sqrt(C))
BN_EPS = 1e-5


def _sig(z):
    return 1.0 / (1.0 + jnp.exp(-z))


def _node_tables_body(x_ref, wq, bq, wk, bk, wv, bv, wku1, wm1,
                      q_o, ka_o, kb_o, va_o, vb_o):
    xb = x_ref[...]
    q = jnp.dot(xb, wq[...], preferred_element_type=jnp.float32) + bq[...]
    k = jnp.dot(xb, wk[...], preferred_element_type=jnp.float32) + bk[...]
    v = jnp.dot(xb, wv[...], preferred_element_type=jnp.float32) + bv[...]
    q_o[...] = q
    ka_o[...] = jnp.dot(k, wku1[0:C, :], preferred_element_type=jnp.float32)
    kb_o[...] = jnp.dot(k, wku1[C:2 * C, :], preferred_element_type=jnp.float32)
    va_o[...] = jnp.dot(v, wm1[0:C, :], preferred_element_type=jnp.float32)
    vb_o[...] = jnp.dot(v, wm1[C:2 * C, :], preferred_element_type=jnp.float32)


def _node_tables(x, p):
    Bn = 2000
    full = lambda shape: pl.BlockSpec(shape, lambda i: (0,) * len(shape))
    return pl.pallas_call(
        _node_tables_body,
        grid=(N // Bn,),
        in_specs=[
            pl.BlockSpec((Bn, D), lambda i: (i, 0)),
            full((D, C)), full((1, C)),
            full((D, C)), full((1, C)),
            full((D, C)), full((1, C)),
            full((3 * C, C)), full((3 * C, C)),
        ],
        out_specs=[pl.BlockSpec((Bn, C), lambda i: (i, 0))] * 5,
        out_shape=[jax.ShapeDtypeStruct((N, C), jnp.float32)] * 5,
    )(x, p['Wq'], p['bq'].reshape(1, C), p['Wk'], p['bk'].reshape(1, C),
      p['Wv'], p['bv'].reshape(1, C), p['Wku1'], p['Wm1'])


def _gather_k_body(dst_h, src_h, q_h, ka_h, kb_h, gq_h, ghk_h,
                   di0, si0, bq0, bhk0, qsem0, ksem0, osem0,
                   di1, si1, bq1, bhk1, qsem1, ksem1, osem1):
    wid = lax.axis_index("s") * NC + lax.axis_index("c")
    base = wid * EPW
    nch = EPW // GB
    sets = ((di0, si0, bq0, bhk0, qsem0, ksem0, osem0),
            (di1, si1, bq1, bhk1, qsem1, ksem1, osem1))

    def stage1(j, st):
        di, si, bq, bhk, qsem, ksem, osem = st
        off = base + j * GB

        @pl.when(j >= 2)
        def _():
            pltpu.make_async_copy(bq, gq_h.at[pl.ds(off, GB)], osem).wait()
            pltpu.make_async_copy(bhk, ghk_h.at[pl.ds(off, GB)], osem).wait()

        pltpu.sync_copy(dst_h.at[pl.ds(off, GB)], di)
        pltpu.sync_copy(src_h.at[pl.ds(off, GB)], si)
        pltpu.async_copy(q_h.at[di], bq, qsem)
        pltpu.async_copy(ka_h.at[di], bhk, ksem)

    def finish(c, s):
        di, si, bq, bhk, qsem, ksem, osem = sets[s]
        off = base + c * GB
        pltpu.make_async_copy(ka_h.at[pl.ds(0, GB)], bhk, ksem).wait()
        kb_cp = pltpu.async_copy(kb_h.at[si], bhk, ksem, add=True)

        @pl.when(c + 1 < nch)
        def _():
            stage1(c + 1, sets[1 - s])

        pltpu.make_async_copy(q_h.at[pl.ds(0, GB)], bq, qsem).wait()
        pltpu.async_copy(bq, gq_h.at[pl.ds(off, GB)], osem)
        kb_cp.wait()
        pltpu.async_copy(bhk, ghk_h.at[pl.ds(off, GB)], osem)

    stage1(0, sets[0])

    def pair(p, carry):
        finish(2 * p, 0)
        finish(2 * p + 1, 1)
        return carry

    lax.fori_loop(0, nch // 2, pair, 0)
    for s in (0, 1):
        di, si, bq, bhk, qsem, ksem, osem = sets[s]
        pltpu.make_async_copy(bq, gq_h.at[pl.ds(base, GB)], osem).wait()
        pltpu.make_async_copy(bhk, ghk_h.at[pl.ds(base, GB)], osem).wait()


def _gather_m_body(dst_h, src_h, va_h, vb_h, ghm_h,
                   di0, si0, bhm0, gsem0, osem0,
                   di1, si1, bhm1, gsem1, osem1):
    wid = lax.axis_index("s") * NC + lax.axis_index("c")
    base = wid * EPW
    nch = EPW // GB
    sets = ((di0, si0, bhm0, gsem0, osem0),
            (di1, si1, bhm1, gsem1, osem1))

    def stage1(j, st):
        di, si, bhm, gsem, osem = st
        off = base + j * GB

        @pl.when(j >= 2)
        def _():
            pltpu.make_async_copy(bhm, ghm_h.at[pl.ds(off, GB)], osem).wait()

        pltpu.sync_copy(dst_h.at[pl.ds(off, GB)], di)
        pltpu.sync_copy(src_h.at[pl.ds(off, GB)], si)
        pltpu.async_copy(va_h.at[di], bhm, gsem)

    def finish(c, s):
        di, si, bhm, gsem, osem = sets[s]
        off = base + c * GB
        pltpu.make_async_copy(va_h.at[pl.ds(0, GB)], bhm, gsem).wait()
        vb_cp = pltpu.async_copy(vb_h.at[si], bhm, gsem, add=True)

        @pl.when(c + 1 < nch)
        def _():
            stage1(c + 1, sets[1 - s])

        vb_cp.wait()
        pltpu.async_copy(bhm, ghm_h.at[pl.ds(off, GB)], osem)

    stage1(0, sets[0])

    def pair(p, carry):
        finish(2 * p, 0)
        finish(2 * p + 1, 1)
        return carry

    lax.fori_loop(0, nch // 2, pair, 0)
    for s in (0, 1):
        di, si, bhm, gsem, osem = sets[s]
        pltpu.make_async_copy(bhm, ghm_h.at[pl.ds(base, GB)], osem).wait()


def _sc_gather_k(dst, src, q, ka, kb):
    mesh = plsc.VectorSubcoreMesh(core_axis_name="c", subcore_axis_name="s",
                                  num_cores=NC, num_subcores=NS)
    set_scratch = [
        pltpu.VMEM((GB,), jnp.int32),
        pltpu.VMEM((GB,), jnp.int32),
        pltpu.VMEM((GB, C), jnp.float32),
        pltpu.VMEM((GB, C), jnp.float32),
        pltpu.SemaphoreType.DMA,
        pltpu.SemaphoreType.DMA,
        pltpu.SemaphoreType.DMA,
    ]
    fn = pl.kernel(
        _gather_k_body,
        out_type=[jax.ShapeDtypeStruct((E, C), jnp.float32)] * 2,
        mesh=mesh,
        scratch_types=set_scratch + set_scratch,
    )
    return fn(dst, src, q, ka, kb)


def _sc_gather_m(dst, src, va, vb):
    mesh = plsc.VectorSubcoreMesh(core_axis_name="c", subcore_axis_name="s",
                                  num_cores=NC, num_subcores=NS)
    set_scratch = [
        pltpu.VMEM((GB,), jnp.int32),
        pltpu.VMEM((GB,), jnp.int32),
        pltpu.VMEM((GB, C), jnp.float32),
        pltpu.SemaphoreType.DMA,
        pltpu.SemaphoreType.DMA,
    ]
    fn = pl.kernel(
        _gather_m_body,
        out_type=jax.ShapeDtypeStruct((E, C), jnp.float32),
        mesh=mesh,
        scratch_types=set_scratch + set_scratch,
    )
    return fn(dst, src, va, vb)


def _pass1_body(ghk, gq, eat, wek, ck, wku2, bku2, alpha_o, stats_o):
    i = pl.program_id(0)

    eak = lax.dot_general(eat[...], wek[...], (((0,), (0,)), ((), ())),
                          preferred_element_type=jnp.float32)
    h = ghk[...] + eak + ck[...]
    h = h * _sig(h)
    kj = jnp.dot(h, wku2[...], preferred_element_type=jnp.float32) + bku2[...]
    alpha = gq[...] * kj * INV_SQRT_C
    alpha_o[...] = alpha.astype(jnp.bfloat16)

    @pl.when(i == 0)
    def _():
        stats_o[...] = jnp.zeros_like(stats_o)

    s1 = jnp.sum(alpha, axis=0, keepdims=True)
    s2 = jnp.sum(alpha * alpha, axis=0, keepdims=True)
    stats_o[0:1, :] += s1
    stats_o[1:2, :] += s2


def _pass1(ghk, gq, eat, wek, ck, wku2, bku2):
    full = lambda shape: pl.BlockSpec(shape, lambda i: (0,) * len(shape))
    return pl.pallas_call(
        _pass1_body,
        grid=(E // BE,),
        in_specs=[
            pl.BlockSpec((BE, C), lambda i: (i, 0)),
            pl.BlockSpec((BE, C), lambda i: (i, 0)),
            pl.BlockSpec((ED, BE), lambda i: (0, i)),
            full((ED, C)), full((1, C)), full((C, C)), full((1, C)),
        ],
        out_specs=[pl.BlockSpec((BE, C), lambda i: (i, 0)),
                   pl.BlockSpec((8, C), lambda i: (0, 0))],
        out_shape=[jax.ShapeDtypeStruct((E, C), jnp.bfloat16),
                   jax.ShapeDtypeStruct((8, C), jnp.float32)],
    )(ghk, gq, eat, wek, ck, wku2, bku2)


def _pass2_body(alpha, ghm, eat, stats, gatt, batt, wem, cm, wm2, bm2,
                gmsg_o):
    mu = stats[0:1, :] * (1.0 / E)
    ex2 = stats[1:2, :] * (1.0 / E)
    var = ex2 - mu * mu
    inv = gatt[...] * lax.rsqrt(var + BN_EPS)
    shift = batt[...] - mu * inv
    gate = _sig(alpha[...].astype(jnp.float32) * inv + shift)

    eam = lax.dot_general(eat[...], wem[...], (((0,), (0,)), ((), ())),
                          preferred_element_type=jnp.float32)
    h = ghm[...] + eam + cm[...]
    h = h * _sig(h)
    msg = jnp.dot(h, wm2[...], preferred_element_type=jnp.float32) + bm2[...]
    gmsg_o[...] = msg * gate


def _pass2(alpha, ghm, eat, stats, gatt, batt, wem, cm, wm2, bm2):
    full = lambda shape: pl.BlockSpec(shape, lambda i: (0,) * len(shape))
    return pl.pallas_call(
        _pass2_body,
        grid=(E // BE,),
        in_specs=[
            pl.BlockSpec((BE, C), lambda i: (i, 0)),
            pl.BlockSpec((BE, C), lambda i: (i, 0)),
            pl.BlockSpec((ED, BE), lambda i: (0, i)),
            full((8, C)), full((1, C)), full((1, C)),
            full((ED, C)), full((1, C)), full((C, C)), full((1, C)),
        ],
        out_specs=pl.BlockSpec((BE, C), lambda i: (i, 0)),
        out_shape=jax.ShapeDtypeStruct((E, C), jnp.float32),
    )(alpha, ghm, eat, stats, gatt, batt, wem, cm, wm2, bm2)


def _sc_scatter_body(dst_h, gmsg_h, zeros_h, parts_h,
                     idx0, dat0, lsem0, ssem0,
                     idx1, dat1, lsem1, ssem1, acc):
    cid = lax.axis_index("c")
    sid = lax.axis_index("s")
    wid = sid * NC + cid
    base = wid * EPW
    nch = EPW // SB
    r0 = sid * INIT_ROWS
    sets = ((idx0, dat0, lsem0, ssem0), (idx1, dat1, lsem1, ssem1))

    @pl.when(sid < N // INIT_ROWS)
    def _():
        pltpu.sync_copy(zeros_h.at[pl.ds(r0, INIT_ROWS)],
                        acc.at[pl.ds(r0, INIT_ROWS)])
    plsc.subcore_barrier()

    def stage1(j, st):
        idxb, datab, lsem, ssem = st

        @pl.when(j >= 2)
        def _():
            pltpu.make_async_copy(datab, acc.at[pl.ds(0, SB)], ssem).wait()

        off = base + j * SB
        pltpu.async_copy(dst_h.at[pl.ds(off, SB)], idxb, lsem)
        pltpu.async_copy(gmsg_h.at[pl.ds(off, SB)], datab, lsem)

    def finish(c, s):
        idxb, datab, lsem, ssem = sets[s]
        pltpu.make_async_copy(dst_h.at[pl.ds(0, SB)], idxb, lsem).wait()
        pltpu.make_async_copy(gmsg_h.at[pl.ds(0, SB)], datab, lsem).wait()
        pltpu.async_copy(datab, acc.at[idxb], ssem, add=True)

        @pl.when(c + 1 < nch)
        def _():
            stage1(c + 1, sets[1 - s])

    stage1(0, sets[0])

    def pair(p, carry):
        finish(2 * p, 0)
        finish(2 * p + 1, 1)
        return carry

    lax.fori_loop(0, nch // 2, pair, 0)
    if nch % 2:
        finish(nch - 1, (nch - 1) % 2)
    for s in (0, 1):
        idxb, datab, lsem, ssem = sets[s]
        pltpu.make_async_copy(datab, acc.at[pl.ds(0, SB)], ssem).wait()
    plsc.subcore_barrier()

    @pl.when(sid < N // INIT_ROWS)
    def _():
        pltpu.sync_copy(acc.at[pl.ds(r0, INIT_ROWS)],
                        parts_h.at[cid, pl.ds(r0, INIT_ROWS)])


def _sc_scatter(dst, gmsg, zeros):
    mesh = plsc.VectorSubcoreMesh(core_axis_name="c", subcore_axis_name="s",
                                  num_cores=NC, num_subcores=NS)
    set_scratch = [
        pltpu.VMEM((SB,), jnp.int32),
        pltpu.VMEM((SB, C), jnp.float32),
        pltpu.SemaphoreType.DMA,
        pltpu.SemaphoreType.DMA,
    ]
    fn = pl.kernel(
        _sc_scatter_body,
        out_type=jax.ShapeDtypeStruct((NC, N, C), jnp.float32),
        mesh=mesh,
        scratch_types=set_scratch + set_scratch + [
            pltpu.VMEM_SHARED((N, C), jnp.float32),
        ],
    )
    return fn(dst, gmsg, zeros)


def _final_body(parts, x_ref, wc, bc, gbn, bbn, out_o):
    agg = parts[0] + parts[1]
    o = jnp.dot(agg, wc[...], preferred_element_type=jnp.float32) + bc[...]
    mu = jnp.sum(o, axis=0, keepdims=True) * (1.0 / N)
    ex2 = jnp.sum(o * o, axis=0, keepdims=True) * (1.0 / N)
    var = ex2 - mu * mu
    inv = gbn[...] * lax.rsqrt(var + BN_EPS)
    z = x_ref[...] + o * inv + (bbn[...] - mu * inv)
    out_o[...] = jnp.maximum(z, 0.0) + jnp.log(1.0 + jnp.exp(-jnp.abs(z)))


def _final(parts, x, wc, bc, gbn, bbn):
    return pl.pallas_call(
        _final_body,
        out_shape=jax.ShapeDtypeStruct((N, C), jnp.float32),
    )(parts, x, wc, bc.reshape(1, C), gbn.reshape(1, C), bbn.reshape(1, C))


def kernel(x, edge_index, edge_attr, params):
    p = params
    ei = edge_index.astype(jnp.int32)
    src = ei[0]
    dst = ei[1]

    wek = p['We'] @ p['Wku1'][2 * C:]
    ck = (p['bku1'] + p['be'] @ p['Wku1'][2 * C:]).reshape(1, C)
    wem = p['We'] @ p['Wm1'][2 * C:]
    cm = (p['bm1'] + p['be'] @ p['Wm1'][2 * C:]).reshape(1, C)

    eat = edge_attr.T
    q, ka, kb, va, vb = _node_tables(x, p)
    gq, ghk = _sc_gather_k(dst, src, q, ka, kb)
    ghm = _sc_gather_m(dst, src, va, vb)
    alpha, stats = _pass1(ghk, gq, eat, wek, ck, p['Wku2'],
                          p['bku2'].reshape(1, C))
    gmsg = _pass2(alpha, ghm, eat, stats,
                  p['g_att'].reshape(1, C), p['b_att'].reshape(1, C),
                  wem, cm, p['Wm2'], p['bm2'].reshape(1, C))
    zeros = jnp.zeros((N, C), jnp.float32)
    parts = _sc_scatter(dst, gmsg, zeros)
    return _final(parts, x, p['Wc'], p['bc'], p['g_bn'], p['b_bn'])
